# baseline (speedup 1.0000x reference)
import functools

import jax
import jax.numpy as jnp
from jax import lax
from jax.experimental import pallas as pl
from jax.experimental.pallas import tpu as pltpu
from jax.experimental.pallas import tpu_sc as plsc

N = 50000
NP = 50176
E = 800000
EP = 819200
G = 256
GACC = 272
EPS = 1e-5
NPB = 6272
EPB = 6400
f32 = jnp.float32

_mesh = lambda: plsc.VectorSubcoreMesh(core_axis_name="c", subcore_axis_name="s")
_SC_PARAMS = pltpu.CompilerParams(use_tc_tiling_on_sc=False)


@functools.lru_cache(maxsize=None)
def _sc_count(ep, k, ib, acc_rows):
    n_chunks = ep // (32 * k)
    n_iters = n_chunks // ib
    stripe = acc_rows // 16

    @functools.partial(
        pl.kernel, mesh=_mesh(), compiler_params=_SC_PARAMS,
        out_type=jax.ShapeDtypeStruct((2, acc_rows, 32), f32),
        scratch_types=[
            pltpu.VMEM((ib, k), jnp.int32),
            pltpu.VMEM((k, 32), f32),
            pltpu.VMEM_SHARED((acc_rows, 32), f32),
            pltpu.SemaphoreType.DMA,
        ],
    )
    def body(sidx2, ones, zeros, out, si2, ones_v, acc, sem):
        c = lax.axis_index("c")
        s = lax.axis_index("s")
        pltpu.sync_copy(zeros.at[pl.ds(s * stripe, stripe)],
                        acc.at[pl.ds(s * stripe, stripe)])
        pltpu.sync_copy(ones.at[pl.ds(0, k)], ones_v)
        plsc.subcore_barrier()
        chunk0 = (c * 16 + s) * n_chunks

        @pl.loop(0, n_iters)
        def _(it):
            blk = chunk0 + it * ib
            pltpu.sync_copy(sidx2.at[pl.ds(blk, ib)], si2)
            cps = [pltpu.async_copy(ones_v, acc.at[si2.at[b]], sem, add=True)
                   for b in range(ib)]
            for cp in cps:
                cp.wait()

        plsc.subcore_barrier()
        pltpu.sync_copy(acc.at[pl.ds(s * stripe, stripe)],
                        out.at[c, pl.ds(s * stripe, stripe)])

    return body


@functools.lru_cache(maxsize=None)
def _sc_gather_segsum(ep, k, ib, acc_rows):
    n_chunks = ep // (32 * k)
    n_iters = n_chunks // ib
    stripe = acc_rows // 16

    @functools.partial(
        pl.kernel, mesh=_mesh(), compiler_params=_SC_PARAMS,
        out_type=jax.ShapeDtypeStruct((2, acc_rows, 32), f32),
        scratch_types=[
            pltpu.VMEM((ib, k), jnp.int32),
            pltpu.VMEM((ib, k), jnp.int32),
            pltpu.VMEM((ib, k, 32), f32),
            pltpu.VMEM_SHARED((acc_rows, 32), f32),
            pltpu.SemaphoreType.DMA,
            pltpu.SemaphoreType.DMA,
        ],
    )
    def body(table, gidx2, sidx2, zeros, out, gi2, si2, rows, acc, semg, sems):
        c = lax.axis_index("c")
        s = lax.axis_index("s")
        pltpu.sync_copy(zeros.at[pl.ds(s * stripe, stripe)],
                        acc.at[pl.ds(s * stripe, stripe)])
        plsc.subcore_barrier()
        chunk0 = (c * 16 + s) * n_chunks

        @pl.loop(0, n_iters)
        def _(it):
            blk = chunk0 + it * ib
            pltpu.sync_copy(gidx2.at[pl.ds(blk, ib)], gi2)
            pltpu.sync_copy(sidx2.at[pl.ds(blk, ib)], si2)
            gs = [pltpu.async_copy(table.at[gi2.at[b]], rows.at[b], semg)
                  for b in range(ib)]
            scs = []
            for b in range(ib):
                gs[b].wait()
                scs.append(pltpu.async_copy(rows.at[b], acc.at[si2.at[b]],
                                            sems, add=True))
            for cp in scs:
                cp.wait()

        plsc.subcore_barrier()
        pltpu.sync_copy(acc.at[pl.ds(s * stripe, stripe)],
                        out.at[c, pl.ds(s * stripe, stripe)])

    return body


@functools.lru_cache(maxsize=None)
def _sc_linear_segsum(ep, k, ib, acc_rows):
    n_chunks = ep // (32 * k)
    n_iters = n_chunks // ib
    stripe = acc_rows // 16

    @functools.partial(
        pl.kernel, mesh=_mesh(), compiler_params=_SC_PARAMS,
        out_type=jax.ShapeDtypeStruct((2, acc_rows, 32), f32),
        scratch_types=[
            pltpu.VMEM((ib, k), jnp.int32),
            pltpu.VMEM((ib * k, 32), f32),
            pltpu.VMEM_SHARED((acc_rows, 32), f32),
            pltpu.SemaphoreType.DMA,
        ],
    )
    def body(rows2d, sidx2, zeros, out, si2, rows_v, acc, sem):
        c = lax.axis_index("c")
        s = lax.axis_index("s")
        pltpu.sync_copy(zeros.at[pl.ds(s * stripe, stripe)],
                        acc.at[pl.ds(s * stripe, stripe)])
        plsc.subcore_barrier()
        chunk0 = (c * 16 + s) * n_chunks

        @pl.loop(0, n_iters)
        def _(it):
            blk = chunk0 + it * ib
            pltpu.sync_copy(sidx2.at[pl.ds(blk, ib)], si2)
            pltpu.sync_copy(rows2d.at[pl.ds(blk * k, ib * k)], rows_v)
            cps = [pltpu.async_copy(rows_v.at[pl.ds(b * k, k)],
                                    acc.at[si2.at[b]], sem, add=True)
                   for b in range(ib)]
            for cp in cps:
                cp.wait()

        plsc.subcore_barrier()
        pltpu.sync_copy(acc.at[pl.ds(s * stripe, stripe)],
                        out.at[c, pl.ds(s * stripe, stripe)])

    return body


@functools.lru_cache(maxsize=None)
def _sc_gather2(ep, k, ib):
    n_chunks = ep // (32 * k)
    n_iters = n_chunks // ib

    @functools.partial(
        pl.kernel, mesh=_mesh(), compiler_params=_SC_PARAMS,
        out_type=(jax.ShapeDtypeStruct((ep, 128), jnp.bfloat16),
                  jax.ShapeDtypeStruct((ep, 128), jnp.bfloat16)),
        scratch_types=[
            pltpu.VMEM((ib, k), jnp.int32),
            pltpu.VMEM((ib, k), jnp.int32),
            pltpu.VMEM((ib, k, 128), jnp.bfloat16),
            pltpu.VMEM((ib, k, 128), jnp.bfloat16),
            pltpu.SemaphoreType.DMA,
            pltpu.SemaphoreType.DMA,
        ],
    )
    def body(tA, tB, idxA2, idxB2, outA, outB, ia2, ib2, bufA, bufB,
             semg, semw):
        c = lax.axis_index("c")
        s = lax.axis_index("s")
        chunk0 = (c * 16 + s) * n_chunks

        @pl.loop(0, n_iters)
        def _(it):
            blk = chunk0 + it * ib
            pltpu.sync_copy(idxA2.at[pl.ds(blk, ib)], ia2)
            pltpu.sync_copy(idxB2.at[pl.ds(blk, ib)], ib2)
            gs = []
            for b in range(ib):
                gs.append(pltpu.async_copy(tA.at[ia2.at[b]], bufA.at[b], semg))
                gs.append(pltpu.async_copy(tB.at[ib2.at[b]], bufB.at[b], semg))
            ws = []
            for b in range(ib):
                gs[2 * b].wait()
                ws.append(pltpu.async_copy(
                    bufA.at[b], outA.at[pl.ds((blk + b) * k, k)], semw))
                gs[2 * b + 1].wait()
                ws.append(pltpu.async_copy(
                    bufB.at[b], outB.at[pl.ds((blk + b) * k, k)], semw))
            for cp in ws:
                cp.wait()

    return body


def _t0_embed(x_p, WeT, be):
    def body(x_ref, w_ref, b_ref, h0_ref, h1_ref):
        h = jnp.dot(x_ref[...], w_ref[...], preferred_element_type=f32) + b_ref[...]
        h0_ref[...] = h[:, :32]
        h1_ref[...] = h[:, 32:]

    return pl.pallas_call(
        body,
        grid=(NP // NPB,),
        in_specs=[
            pl.BlockSpec((NPB, 92), lambda i: (i, 0)),
            pl.BlockSpec((92, 64), lambda i: (0, 0)),
            pl.BlockSpec((1, 64), lambda i: (0, 0)),
        ],
        out_specs=[pl.BlockSpec((NPB, 32), lambda i: (i, 0))] * 2,
        out_shape=[jax.ShapeDtypeStruct((NP, 32), f32)] * 2,
    )(x_p, WeT, be)


def _t1a(hs0, hs1, cnt_h, ha_p, W1, b1):
    nb = NP // NPB

    def body(hs0_ref, hs1_ref, cnt_ref, ha_ref, w_ref, b_ref,
             z_ref, st_ref, acc):
        i = pl.program_id(0)
        c = cnt_ref[0, :, :1] + cnt_ref[1, :, :1]
        r = 1.0 / jnp.maximum(c, 1.0)
        hm0 = (hs0_ref[0] + hs0_ref[1]) * r
        hm1 = (hs1_ref[0] + hs1_ref[1]) * r
        msg = jnp.concatenate([hm0, hm1, ha_ref[...]], axis=1)
        z = jnp.dot(msg, w_ref[...], preferred_element_type=f32) + b_ref[...]
        z_ref[...] = z
        rows = i * NPB + lax.broadcasted_iota(jnp.int32, (NPB, 1), 0)
        zm = jnp.where(rows < N, z, 0.0)
        s1 = jnp.sum(zm, axis=0)
        s2 = jnp.sum(zm * zm, axis=0)
        upd = jnp.concatenate(
            [s1[None, :], s2[None, :], jnp.zeros((6, 70), f32)], axis=0)

        @pl.when(i == 0)
        def _():
            acc[...] = jnp.zeros_like(acc)

        acc[...] += upd

        @pl.when(i == nb - 1)
        def _():
            st_ref[...] = acc[...]

    return pl.pallas_call(
        body,
        grid=(nb,),
        in_specs=[
            pl.BlockSpec((2, NPB, 32), lambda i: (0, i, 0)),
            pl.BlockSpec((2, NPB, 32), lambda i: (0, i, 0)),
            pl.BlockSpec((2, NPB, 32), lambda i: (0, i, 0)),
            pl.BlockSpec((NPB, 35), lambda i: (i, 0)),
            pl.BlockSpec((99, 70), lambda i: (0, 0)),
            pl.BlockSpec((1, 70), lambda i: (0, 0)),
        ],
        out_specs=[
            pl.BlockSpec((NPB, 70), lambda i: (i, 0)),
            pl.BlockSpec((8, 70), lambda i: (0, 0)),
        ],
        out_shape=[
            jax.ShapeDtypeStruct((NP, 70), f32),
            jax.ShapeDtypeStruct((8, 70), f32),
        ],
        scratch_shapes=[pltpu.VMEM((8, 70), f32)],
    )(hs0, hs1, cnt_h, ha_p, W1, b1)


def _t1b(z, st, g1, be1, h0, h1, WA, bA, WB):
    def body(z_ref, st_ref, g_ref, be_ref, h0_ref, h1_ref,
             wa_ref, ba_ref, wb_ref, afc_ref, bfc_ref):
        mean = st_ref[0, :] * (1.0 / N)
        var = st_ref[1, :] * (1.0 / N) - mean * mean
        scale = g_ref[0, :] * lax.rsqrt(var + EPS)
        zn = (z_ref[...] - mean[None, :]) * scale[None, :] + be_ref[...]
        ha = jax.nn.sigmoid(zn[:, :35]) * jax.nn.softplus(zn[:, 35:70])
        bfc_ref[...] = jnp.dot(ha, wb_ref[...],
                               preferred_element_type=f32).astype(jnp.bfloat16)
        h = jnp.concatenate([h0_ref[...], h1_ref[...]], axis=1)
        afc_ref[...] = (jnp.dot(h, wa_ref[...], preferred_element_type=f32)
                        + ba_ref[...]).astype(jnp.bfloat16)

    return pl.pallas_call(
        body,
        grid=(NP // NPB,),
        in_specs=[
            pl.BlockSpec((NPB, 70), lambda i: (i, 0)),
            pl.BlockSpec((8, 70), lambda i: (0, 0)),
            pl.BlockSpec((1, 70), lambda i: (0, 0)),
            pl.BlockSpec((1, 70), lambda i: (0, 0)),
            pl.BlockSpec((NPB, 32), lambda i: (i, 0)),
            pl.BlockSpec((NPB, 32), lambda i: (i, 0)),
            pl.BlockSpec((64, 128), lambda i: (0, 0)),
            pl.BlockSpec((1, 128), lambda i: (0, 0)),
            pl.BlockSpec((35, 128), lambda i: (0, 0)),
        ],
        out_specs=[
            pl.BlockSpec((NPB, 128), lambda i: (i, 0)),
            pl.BlockSpec((NPB, 128), lambda i: (i, 0)),
        ],
        out_shape=[jax.ShapeDtypeStruct((NP, 128), jnp.bfloat16)] * 2,
    )(z, st, g1, be1, h0, h1, WA, bA, WB)


def _t2_gate(An, Bh):
    def body(a_ref, b_ref, m0_ref, m1_ref):
        e = a_ref[...].astype(f32) + b_ref[...].astype(f32)
        m = jax.nn.sigmoid(e[:, :64]) * jax.nn.softplus(e[:, 64:])
        m0_ref[...] = m[:, :32]
        m1_ref[...] = m[:, 32:]

    return pl.pallas_call(
        body,
        grid=(EP // EPB,),
        in_specs=[
            pl.BlockSpec((EPB, 128), lambda i: (i, 0)),
            pl.BlockSpec((EPB, 128), lambda i: (i, 0)),
        ],
        out_specs=[pl.BlockSpec((EPB, 32), lambda i: (i, 0))] * 2,
        out_shape=[jax.ShapeDtypeStruct((EP, 32), f32)] * 2,
    )(An, Bh)


def _t3a(ns0, ns1, cnt_n):
    nb = NP // NPB

    def body(ns0_ref, ns1_ref, cnt_ref, nm_ref, st_ref, acc):
        i = pl.program_id(0)
        c = cnt_ref[0, :, :1] + cnt_ref[1, :, :1]
        r = 1.0 / jnp.maximum(c, 1.0)
        nm = jnp.concatenate([(ns0_ref[0] + ns0_ref[1]) * r,
                              (ns1_ref[0] + ns1_ref[1]) * r], axis=1)
        nm_ref[...] = nm
        rows = i * NPB + lax.broadcasted_iota(jnp.int32, (NPB, 1), 0)
        nmm = jnp.where(rows < N, nm, 0.0)
        s1 = jnp.sum(nmm, axis=0)
        s2 = jnp.sum(nmm * nmm, axis=0)
        upd = jnp.concatenate(
            [s1[None, :], s2[None, :], jnp.zeros((6, 64), f32)], axis=0)

        @pl.when(i == 0)
        def _():
            acc[...] = jnp.zeros_like(acc)

        acc[...] += upd

        @pl.when(i == nb - 1)
        def _():
            st_ref[...] = acc[...]

    return pl.pallas_call(
        body,
        grid=(nb,),
        in_specs=[
            pl.BlockSpec((2, NPB, 32), lambda i: (0, i, 0)),
            pl.BlockSpec((2, NPB, 32), lambda i: (0, i, 0)),
            pl.BlockSpec((2, NPB, 32), lambda i: (0, i, 0)),
        ],
        out_specs=[
            pl.BlockSpec((NPB, 64), lambda i: (i, 0)),
            pl.BlockSpec((8, 64), lambda i: (0, 0)),
        ],
        out_shape=[
            jax.ShapeDtypeStruct((NP, 64), f32),
            jax.ShapeDtypeStruct((8, 64), f32),
        ],
        scratch_shapes=[pltpu.VMEM((8, 64), f32)],
    )(ns0, ns1, cnt_n)


def _t3b(nm, st, go, bo, h0, h1):
    def body(nm_ref, st_ref, g_ref, b_ref, h0_ref, h1_ref, o0_ref, o1_ref):
        mean = st_ref[0, :] * (1.0 / N)
        var = st_ref[1, :] * (1.0 / N) - mean * mean
        scale = g_ref[0, :] * lax.rsqrt(var + EPS)
        y = (nm_ref[...] - mean[None, :]) * scale[None, :] + b_ref[...]
        h = jnp.concatenate([h0_ref[...], h1_ref[...]], axis=1)
        hn = jax.nn.relu(jax.nn.softplus(y + h))
        o0_ref[...] = hn[:, :32]
        o1_ref[...] = hn[:, 32:]

    return pl.pallas_call(
        body,
        grid=(NP // NPB,),
        in_specs=[
            pl.BlockSpec((NPB, 64), lambda i: (i, 0)),
            pl.BlockSpec((8, 64), lambda i: (0, 0)),
            pl.BlockSpec((1, 64), lambda i: (0, 0)),
            pl.BlockSpec((1, 64), lambda i: (0, 0)),
            pl.BlockSpec((NPB, 32), lambda i: (i, 0)),
            pl.BlockSpec((NPB, 32), lambda i: (i, 0)),
        ],
        out_specs=[pl.BlockSpec((NPB, 32), lambda i: (i, 0))] * 2,
        out_shape=[jax.ShapeDtypeStruct((NP, 32), f32)] * 2,
    )(nm, st, go, bo, h0, h1)


def _t4_head(gs0, gs1, cnt_g, W2, b2, Wo, bo):
    def body(gs0_ref, gs1_ref, cnt_ref, w2_ref, b2_ref, wo_ref, bo_ref, o_ref):
        c = cnt_ref[0, :, :1] + cnt_ref[1, :, :1]
        r = 1.0 / jnp.maximum(c, 1.0)
        g = jnp.concatenate([(gs0_ref[0] + gs0_ref[1]) * r,
                             (gs1_ref[0] + gs1_ref[1]) * r], axis=1)
        t = jax.nn.softplus(
            jnp.dot(g, w2_ref[...], preferred_element_type=f32) + b2_ref[...])
        o_ref[...] = jnp.dot(t, wo_ref[...],
                             preferred_element_type=f32) + bo_ref[...]

    return pl.pallas_call(
        body,
        in_specs=[
            pl.BlockSpec((2, GACC, 32), lambda: (0, 0, 0)),
            pl.BlockSpec((2, GACC, 32), lambda: (0, 0, 0)),
            pl.BlockSpec((2, GACC, 32), lambda: (0, 0, 0)),
            pl.BlockSpec((64, 128), lambda: (0, 0)),
            pl.BlockSpec((1, 128), lambda: (0, 0)),
            pl.BlockSpec((128, 128), lambda: (0, 0)),
            pl.BlockSpec((1, 128), lambda: (0, 0)),
        ],
        out_specs=pl.BlockSpec((GACC, 128), lambda: (0, 0)),
        out_shape=jax.ShapeDtypeStruct((GACC, 128), f32),
    )(gs0, gs1, cnt_g, W2, b2, Wo, bo)


def kernel(x, hyperedge_index, hedge_attr, batch, params):
    nidx = hyperedge_index[0]
    hidx = hyperedge_index[1]
    nidx_g = jnp.pad(nidx, (0, EP - E)).reshape(EP // 128, 128)
    nidx_s = jnp.pad(nidx, (0, EP - E),
                     constant_values=N).reshape(EP // 128, 128)
    hidx_g = jnp.pad(hidx, (0, EP - E)).reshape(EP // 128, 128)
    hidx_s = jnp.pad(hidx, (0, EP - E),
                     constant_values=N).reshape(EP // 128, 128)
    batch_s = jnp.pad(batch, (0, NP - N),
                      constant_values=G).reshape(NP // 32, 32)

    x_p = jnp.pad(x, ((0, NP - N), (0, 0)))
    ha_p = jnp.pad(hedge_attr, ((0, NP - N), (0, 0)))
    zeros_np = jnp.zeros((NP, 32), f32)
    ones_e = jnp.ones((128, 32), f32)
    ones_g = jnp.ones((32, 32), f32)

    p = params
    WeT = p['embed']['W'].T
    be = p['embed']['b'][None, :]

    cnt_h = _sc_count(EP, 128, 4, NP)(hidx_s, ones_e, zeros_np)
    cnt_n = _sc_count(EP, 128, 4, NP)(nidx_s, ones_e, zeros_np)
    cnt_g = _sc_count(NP, 32, 7, GACC)(batch_s, ones_g, zeros_np)

    h0, h1 = _t0_embed(x_p, WeT, be)

    for lp in p['layers']:
        W1 = jnp.concatenate([lp['lin_f1']['W'].T, lp['lin_c1']['W'].T], axis=1)
        b1 = jnp.concatenate([lp['lin_f1']['b'], lp['lin_c1']['b']])[None, :]
        g1 = jnp.concatenate([lp['bn_f']['g'], lp['bn_c']['g']])[None, :]
        be1 = jnp.concatenate([lp['bn_f']['b'], lp['bn_c']['b']])[None, :]
        WA = jnp.concatenate([lp['lin_f2']['W'][:, :64].T,
                              lp['lin_c2']['W'][:, :64].T], axis=1)
        bA = jnp.concatenate([lp['lin_f2']['b'], lp['lin_c2']['b']])[None, :]
        WB = jnp.concatenate([lp['lin_f2']['W'][:, 64:].T,
                              lp['lin_c2']['W'][:, 64:].T], axis=1)

        hs0 = _sc_gather_segsum(EP, 128, 4, NP)(h0, nidx_g, hidx_s, zeros_np)
        hs1 = _sc_gather_segsum(EP, 128, 4, NP)(h1, nidx_g, hidx_s, zeros_np)
        z, st = _t1a(hs0, hs1, cnt_h, ha_p, W1, b1)
        Afc, Bfc = _t1b(z, st, g1, be1, h0, h1, WA, bA, WB)
        An, Bh = _sc_gather2(EP, 128, 4)(Afc, Bfc, nidx_g, hidx_g)
        m0, m1 = _t2_gate(An, Bh)
        ns0 = _sc_linear_segsum(EP, 128, 4, NP)(m0, nidx_s, zeros_np)
        ns1 = _sc_linear_segsum(EP, 128, 4, NP)(m1, nidx_s, zeros_np)
        nm, nst = _t3a(ns0, ns1, cnt_n)
        h0, h1 = _t3b(nm, nst, lp['bn_o']['g'][None, :],
                      lp['bn_o']['b'][None, :], h0, h1)

    gs0 = _sc_linear_segsum(NP, 32, 7, GACC)(h0, batch_s, zeros_np)
    gs1 = _sc_linear_segsum(NP, 32, 7, GACC)(h1, batch_s, zeros_np)

    W2 = p['l2']['W'].T
    b2 = p['l2']['b'][None, :]
    Wo = jnp.zeros((128, 128), f32).at[:, :1].set(p['out']['W'].T)
    bo = jnp.broadcast_to(p['out']['b'][None, :], (1, 128))

    out = _t4_head(gs0, gs1, cnt_g, W2, b2, Wo, bo)
    return out[:G, :1]


# speedup vs baseline: 1.1779x; 1.1779x over previous
import functools

import jax
import jax.numpy as jnp
from jax import lax
from jax.experimental import pallas as pl
from jax.experimental.pallas import tpu as pltpu
from jax.experimental.pallas import tpu_sc as plsc

N = 50000
NP = 50176
E = 800000
EP = 819200
G = 256
GACC = 272
EPS = 1e-5
NPB = 6272
EPB = 6400
f32 = jnp.float32

_mesh = lambda: plsc.VectorSubcoreMesh(core_axis_name="c", subcore_axis_name="s")
_SC_PARAMS = pltpu.CompilerParams(use_tc_tiling_on_sc=False)


@functools.lru_cache(maxsize=None)
def _sc_count(ep, k, ib, ibig, acc_rows):
    n_chunks = ep // (32 * k)
    stripe = acc_rows // 16

    @functools.partial(
        pl.kernel, mesh=_mesh(), compiler_params=_SC_PARAMS,
        out_type=jax.ShapeDtypeStruct((2, acc_rows, 32), f32),
        scratch_types=[
            pltpu.VMEM((ibig, k), jnp.int32),
            pltpu.VMEM((k, 32), f32),
            pltpu.VMEM_SHARED((acc_rows, 32), f32),
            pltpu.SemaphoreType.DMA,
        ],
    )
    def body(sidx2, ones, zeros, out, si2, ones_v, acc, sem):
        c = lax.axis_index("c")
        s = lax.axis_index("s")
        pltpu.sync_copy(zeros.at[pl.ds(s * stripe, stripe)],
                        acc.at[pl.ds(s * stripe, stripe)])
        pltpu.sync_copy(ones.at[pl.ds(0, k)], ones_v)
        plsc.subcore_barrier()
        chunk0 = (c * 16 + s) * n_chunks

        @pl.loop(0, n_chunks // ibig)
        def _(ob):
            blk0 = chunk0 + ob * ibig
            pltpu.sync_copy(sidx2.at[pl.ds(blk0, ibig)], si2)

            @pl.loop(0, ibig // ib)
            def _(g):
                cps = [pltpu.async_copy(ones_v, acc.at[si2.at[g * ib + b]],
                                        sem, add=True)
                       for b in range(ib)]
                for cp in cps:
                    cp.wait()

        plsc.subcore_barrier()
        pltpu.sync_copy(acc.at[pl.ds(s * stripe, stripe)],
                        out.at[c, pl.ds(s * stripe, stripe)])

    return body


@functools.lru_cache(maxsize=None)
def _sc_gather_segsum(ep, k, ib, ibig, acc_rows):
    n_chunks = ep // (32 * k)
    stripe = acc_rows // 16

    @functools.partial(
        pl.kernel, mesh=_mesh(), compiler_params=_SC_PARAMS,
        out_type=jax.ShapeDtypeStruct((2, acc_rows, 32), f32),
        scratch_types=[
            pltpu.VMEM((ibig, k), jnp.int32),
            pltpu.VMEM((ibig, k), jnp.int32),
            pltpu.VMEM((ib, k, 32), f32),
            pltpu.VMEM_SHARED((acc_rows, 32), f32),
            pltpu.SemaphoreType.DMA,
            pltpu.SemaphoreType.DMA,
            pltpu.SemaphoreType.DMA,
        ],
    )
    def body(table, gidx2, sidx2, zeros, out, gi2, si2, rows, acc,
             semi, semg, sems):
        c = lax.axis_index("c")
        s = lax.axis_index("s")
        pltpu.sync_copy(zeros.at[pl.ds(s * stripe, stripe)],
                        acc.at[pl.ds(s * stripe, stripe)])
        plsc.subcore_barrier()
        chunk0 = (c * 16 + s) * n_chunks

        @pl.loop(0, n_chunks // ibig)
        def _(ob):
            blk0 = chunk0 + ob * ibig
            cg = pltpu.async_copy(gidx2.at[pl.ds(blk0, ibig)], gi2, semi)
            cs = pltpu.async_copy(sidx2.at[pl.ds(blk0, ibig)], si2, semi)
            cg.wait()
            cs.wait()

            @pl.loop(0, ibig // ib)
            def _(g):
                gs = [pltpu.async_copy(table.at[gi2.at[g * ib + b]],
                                       rows.at[b], semg)
                      for b in range(ib)]
                scs = []
                for b in range(ib):
                    gs[b].wait()
                    scs.append(pltpu.async_copy(
                        rows.at[b], acc.at[si2.at[g * ib + b]],
                        sems, add=True))
                for cp in scs:
                    cp.wait()

        plsc.subcore_barrier()
        pltpu.sync_copy(acc.at[pl.ds(s * stripe, stripe)],
                        out.at[c, pl.ds(s * stripe, stripe)])

    return body


@functools.lru_cache(maxsize=None)
def _sc_linear_segsum(ep, k, ib, ibig, acc_rows):
    n_chunks = ep // (32 * k)
    stripe = acc_rows // 16

    @functools.partial(
        pl.kernel, mesh=_mesh(), compiler_params=_SC_PARAMS,
        out_type=jax.ShapeDtypeStruct((2, acc_rows, 32), f32),
        scratch_types=[
            pltpu.VMEM((ibig, k), jnp.int32),
            pltpu.VMEM((ib * k, 32), f32),
            pltpu.VMEM_SHARED((acc_rows, 32), f32),
            pltpu.SemaphoreType.DMA,
        ],
    )
    def body(rows2d, sidx2, zeros, out, si2, rows_v, acc, sem):
        c = lax.axis_index("c")
        s = lax.axis_index("s")
        pltpu.sync_copy(zeros.at[pl.ds(s * stripe, stripe)],
                        acc.at[pl.ds(s * stripe, stripe)])
        plsc.subcore_barrier()
        chunk0 = (c * 16 + s) * n_chunks

        @pl.loop(0, n_chunks // ibig)
        def _(ob):
            blk0 = chunk0 + ob * ibig
            pltpu.sync_copy(sidx2.at[pl.ds(blk0, ibig)], si2)

            @pl.loop(0, ibig // ib)
            def _(g):
                blk = blk0 + g * ib
                pltpu.sync_copy(rows2d.at[pl.ds(blk * k, ib * k)], rows_v)
                cps = [pltpu.async_copy(rows_v.at[pl.ds(b * k, k)],
                                        acc.at[si2.at[g * ib + b]],
                                        sem, add=True)
                       for b in range(ib)]
                for cp in cps:
                    cp.wait()

        plsc.subcore_barrier()
        pltpu.sync_copy(acc.at[pl.ds(s * stripe, stripe)],
                        out.at[c, pl.ds(s * stripe, stripe)])

    return body


@functools.lru_cache(maxsize=None)
def _sc_gather2(ep, k, ib, ibig):
    n_chunks = ep // (32 * k)

    @functools.partial(
        pl.kernel, mesh=_mesh(), compiler_params=_SC_PARAMS,
        out_type=(jax.ShapeDtypeStruct((ep, 128), f32),
                  jax.ShapeDtypeStruct((ep, 128), f32)),
        scratch_types=[
            pltpu.VMEM((ibig, k), jnp.int32),
            pltpu.VMEM((ibig, k), jnp.int32),
            pltpu.VMEM((ib, k, 128), f32),
            pltpu.VMEM((ib, k, 128), f32),
            pltpu.SemaphoreType.DMA,
            pltpu.SemaphoreType.DMA,
            pltpu.SemaphoreType.DMA,
        ],
    )
    def body(tA, tB, idxA2, idxB2, outA, outB, ia2, ib2, bufA, bufB,
             semi, semg, semw):
        c = lax.axis_index("c")
        s = lax.axis_index("s")
        chunk0 = (c * 16 + s) * n_chunks

        @pl.loop(0, n_chunks // ibig)
        def _(ob):
            blk0 = chunk0 + ob * ibig
            ca = pltpu.async_copy(idxA2.at[pl.ds(blk0, ibig)], ia2, semi)
            cb = pltpu.async_copy(idxB2.at[pl.ds(blk0, ibig)], ib2, semi)
            ca.wait()
            cb.wait()

            @pl.loop(0, ibig // ib)
            def _(g):
                blk = blk0 + g * ib
                gs = []
                for b in range(ib):
                    gs.append(pltpu.async_copy(
                        tA.at[ia2.at[g * ib + b]], bufA.at[b], semg))
                    gs.append(pltpu.async_copy(
                        tB.at[ib2.at[g * ib + b]], bufB.at[b], semg))
                ws = []
                for b in range(ib):
                    gs[2 * b].wait()
                    ws.append(pltpu.async_copy(
                        bufA.at[b], outA.at[pl.ds((blk + b) * k, k)], semw))
                    gs[2 * b + 1].wait()
                    ws.append(pltpu.async_copy(
                        bufB.at[b], outB.at[pl.ds((blk + b) * k, k)], semw))
                for cp in ws:
                    cp.wait()

    return body


def _t0_embed(x_p, WeT, be):
    def body(x_ref, w_ref, b_ref, h0_ref, h1_ref):
        h = jnp.dot(x_ref[...], w_ref[...], preferred_element_type=f32) + b_ref[...]
        h0_ref[...] = h[:, :32]
        h1_ref[...] = h[:, 32:]

    return pl.pallas_call(
        body,
        grid=(NP // NPB,),
        in_specs=[
            pl.BlockSpec((NPB, 92), lambda i: (i, 0)),
            pl.BlockSpec((92, 64), lambda i: (0, 0)),
            pl.BlockSpec((1, 64), lambda i: (0, 0)),
        ],
        out_specs=[pl.BlockSpec((NPB, 32), lambda i: (i, 0))] * 2,
        out_shape=[jax.ShapeDtypeStruct((NP, 32), f32)] * 2,
    )(x_p, WeT, be)


def _t1a(hs0, hs1, cnt_h, ha_p, W1, b1):
    nb = NP // NPB

    def body(hs0_ref, hs1_ref, cnt_ref, ha_ref, w_ref, b_ref,
             z_ref, st_ref, acc):
        i = pl.program_id(0)
        c = cnt_ref[0, :, :1] + cnt_ref[1, :, :1]
        r = 1.0 / jnp.maximum(c, 1.0)
        hm0 = (hs0_ref[0] + hs0_ref[1]) * r
        hm1 = (hs1_ref[0] + hs1_ref[1]) * r
        msg = jnp.concatenate([hm0, hm1, ha_ref[...]], axis=1)
        z = jnp.dot(msg, w_ref[...], preferred_element_type=f32) + b_ref[...]
        z_ref[...] = z
        rows = i * NPB + lax.broadcasted_iota(jnp.int32, (NPB, 1), 0)
        zm = jnp.where(rows < N, z, 0.0)
        s1 = jnp.sum(zm, axis=0)
        s2 = jnp.sum(zm * zm, axis=0)
        upd = jnp.concatenate(
            [s1[None, :], s2[None, :], jnp.zeros((6, 70), f32)], axis=0)

        @pl.when(i == 0)
        def _():
            acc[...] = jnp.zeros_like(acc)

        acc[...] += upd

        @pl.when(i == nb - 1)
        def _():
            st_ref[...] = acc[...]

    return pl.pallas_call(
        body,
        grid=(nb,),
        in_specs=[
            pl.BlockSpec((2, NPB, 32), lambda i: (0, i, 0)),
            pl.BlockSpec((2, NPB, 32), lambda i: (0, i, 0)),
            pl.BlockSpec((2, NPB, 32), lambda i: (0, i, 0)),
            pl.BlockSpec((NPB, 35), lambda i: (i, 0)),
            pl.BlockSpec((99, 70), lambda i: (0, 0)),
            pl.BlockSpec((1, 70), lambda i: (0, 0)),
        ],
        out_specs=[
            pl.BlockSpec((NPB, 70), lambda i: (i, 0)),
            pl.BlockSpec((8, 70), lambda i: (0, 0)),
        ],
        out_shape=[
            jax.ShapeDtypeStruct((NP, 70), f32),
            jax.ShapeDtypeStruct((8, 70), f32),
        ],
        scratch_shapes=[pltpu.VMEM((8, 70), f32)],
    )(hs0, hs1, cnt_h, ha_p, W1, b1)


def _t1b(z, st, g1, be1, h0, h1, WA, bA, WB):
    def body(z_ref, st_ref, g_ref, be_ref, h0_ref, h1_ref,
             wa_ref, ba_ref, wb_ref, afc_ref, bfc_ref):
        mean = st_ref[0, :] * (1.0 / N)
        var = st_ref[1, :] * (1.0 / N) - mean * mean
        scale = g_ref[0, :] * lax.rsqrt(var + EPS)
        zn = (z_ref[...] - mean[None, :]) * scale[None, :] + be_ref[...]
        ha = jax.nn.sigmoid(zn[:, :35]) * jax.nn.softplus(zn[:, 35:70])
        bfc_ref[...] = jnp.dot(ha, wb_ref[...], preferred_element_type=f32)
        h = jnp.concatenate([h0_ref[...], h1_ref[...]], axis=1)
        afc_ref[...] = jnp.dot(h, wa_ref[...],
                               preferred_element_type=f32) + ba_ref[...]

    return pl.pallas_call(
        body,
        grid=(NP // NPB,),
        in_specs=[
            pl.BlockSpec((NPB, 70), lambda i: (i, 0)),
            pl.BlockSpec((8, 70), lambda i: (0, 0)),
            pl.BlockSpec((1, 70), lambda i: (0, 0)),
            pl.BlockSpec((1, 70), lambda i: (0, 0)),
            pl.BlockSpec((NPB, 32), lambda i: (i, 0)),
            pl.BlockSpec((NPB, 32), lambda i: (i, 0)),
            pl.BlockSpec((64, 128), lambda i: (0, 0)),
            pl.BlockSpec((1, 128), lambda i: (0, 0)),
            pl.BlockSpec((35, 128), lambda i: (0, 0)),
        ],
        out_specs=[
            pl.BlockSpec((NPB, 128), lambda i: (i, 0)),
            pl.BlockSpec((NPB, 128), lambda i: (i, 0)),
        ],
        out_shape=[jax.ShapeDtypeStruct((NP, 128), f32)] * 2,
    )(z, st, g1, be1, h0, h1, WA, bA, WB)


def _t2_gate(An, Bh):
    def body(a_ref, b_ref, m0_ref, m1_ref):
        e = a_ref[...] + b_ref[...]
        m = jax.nn.sigmoid(e[:, :64]) * jax.nn.softplus(e[:, 64:])
        m0_ref[...] = m[:, :32]
        m1_ref[...] = m[:, 32:]

    return pl.pallas_call(
        body,
        grid=(EP // EPB,),
        in_specs=[
            pl.BlockSpec((EPB, 128), lambda i: (i, 0)),
            pl.BlockSpec((EPB, 128), lambda i: (i, 0)),
        ],
        out_specs=[pl.BlockSpec((EPB, 32), lambda i: (i, 0))] * 2,
        out_shape=[jax.ShapeDtypeStruct((EP, 32), f32)] * 2,
    )(An, Bh)


def _t3a(ns0, ns1, cnt_n):
    nb = NP // NPB

    def body(ns0_ref, ns1_ref, cnt_ref, nm_ref, st_ref, acc):
        i = pl.program_id(0)
        c = cnt_ref[0, :, :1] + cnt_ref[1, :, :1]
        r = 1.0 / jnp.maximum(c, 1.0)
        nm = jnp.concatenate([(ns0_ref[0] + ns0_ref[1]) * r,
                              (ns1_ref[0] + ns1_ref[1]) * r], axis=1)
        nm_ref[...] = nm
        rows = i * NPB + lax.broadcasted_iota(jnp.int32, (NPB, 1), 0)
        nmm = jnp.where(rows < N, nm, 0.0)
        s1 = jnp.sum(nmm, axis=0)
        s2 = jnp.sum(nmm * nmm, axis=0)
        upd = jnp.concatenate(
            [s1[None, :], s2[None, :], jnp.zeros((6, 64), f32)], axis=0)

        @pl.when(i == 0)
        def _():
            acc[...] = jnp.zeros_like(acc)

        acc[...] += upd

        @pl.when(i == nb - 1)
        def _():
            st_ref[...] = acc[...]

    return pl.pallas_call(
        body,
        grid=(nb,),
        in_specs=[
            pl.BlockSpec((2, NPB, 32), lambda i: (0, i, 0)),
            pl.BlockSpec((2, NPB, 32), lambda i: (0, i, 0)),
            pl.BlockSpec((2, NPB, 32), lambda i: (0, i, 0)),
        ],
        out_specs=[
            pl.BlockSpec((NPB, 64), lambda i: (i, 0)),
            pl.BlockSpec((8, 64), lambda i: (0, 0)),
        ],
        out_shape=[
            jax.ShapeDtypeStruct((NP, 64), f32),
            jax.ShapeDtypeStruct((8, 64), f32),
        ],
        scratch_shapes=[pltpu.VMEM((8, 64), f32)],
    )(ns0, ns1, cnt_n)


def _t3b(nm, st, go, bo, h0, h1):
    def body(nm_ref, st_ref, g_ref, b_ref, h0_ref, h1_ref, o0_ref, o1_ref):
        mean = st_ref[0, :] * (1.0 / N)
        var = st_ref[1, :] * (1.0 / N) - mean * mean
        scale = g_ref[0, :] * lax.rsqrt(var + EPS)
        y = (nm_ref[...] - mean[None, :]) * scale[None, :] + b_ref[...]
        h = jnp.concatenate([h0_ref[...], h1_ref[...]], axis=1)
        hn = jax.nn.relu(jax.nn.softplus(y + h))
        o0_ref[...] = hn[:, :32]
        o1_ref[...] = hn[:, 32:]

    return pl.pallas_call(
        body,
        grid=(NP // NPB,),
        in_specs=[
            pl.BlockSpec((NPB, 64), lambda i: (i, 0)),
            pl.BlockSpec((8, 64), lambda i: (0, 0)),
            pl.BlockSpec((1, 64), lambda i: (0, 0)),
            pl.BlockSpec((1, 64), lambda i: (0, 0)),
            pl.BlockSpec((NPB, 32), lambda i: (i, 0)),
            pl.BlockSpec((NPB, 32), lambda i: (i, 0)),
        ],
        out_specs=[pl.BlockSpec((NPB, 32), lambda i: (i, 0))] * 2,
        out_shape=[jax.ShapeDtypeStruct((NP, 32), f32)] * 2,
    )(nm, st, go, bo, h0, h1)


def _t4_head(gs0, gs1, cnt_g, W2, b2, Wo, bo):
    def body(gs0_ref, gs1_ref, cnt_ref, w2_ref, b2_ref, wo_ref, bo_ref, o_ref):
        c = cnt_ref[0, :, :1] + cnt_ref[1, :, :1]
        r = 1.0 / jnp.maximum(c, 1.0)
        g = jnp.concatenate([(gs0_ref[0] + gs0_ref[1]) * r,
                             (gs1_ref[0] + gs1_ref[1]) * r], axis=1)
        t = jax.nn.softplus(
            jnp.dot(g, w2_ref[...], preferred_element_type=f32) + b2_ref[...])
        o_ref[...] = jnp.dot(t, wo_ref[...],
                             preferred_element_type=f32) + bo_ref[...]

    return pl.pallas_call(
        body,
        in_specs=[
            pl.BlockSpec((2, GACC, 32), lambda: (0, 0, 0)),
            pl.BlockSpec((2, GACC, 32), lambda: (0, 0, 0)),
            pl.BlockSpec((2, GACC, 32), lambda: (0, 0, 0)),
            pl.BlockSpec((64, 128), lambda: (0, 0)),
            pl.BlockSpec((1, 128), lambda: (0, 0)),
            pl.BlockSpec((128, 128), lambda: (0, 0)),
            pl.BlockSpec((1, 128), lambda: (0, 0)),
        ],
        out_specs=pl.BlockSpec((GACC, 128), lambda: (0, 0)),
        out_shape=jax.ShapeDtypeStruct((GACC, 128), f32),
    )(gs0, gs1, cnt_g, W2, b2, Wo, bo)


def kernel(x, hyperedge_index, hedge_attr, batch, params):
    nidx = hyperedge_index[0]
    hidx = hyperedge_index[1]
    nidx_g = jnp.pad(nidx, (0, EP - E)).reshape(EP // 128, 128)
    nidx_s = jnp.pad(nidx, (0, EP - E),
                     constant_values=N).reshape(EP // 128, 128)
    hidx_g = jnp.pad(hidx, (0, EP - E)).reshape(EP // 128, 128)
    hidx_s = jnp.pad(hidx, (0, EP - E),
                     constant_values=N).reshape(EP // 128, 128)
    batch_s = jnp.pad(batch, (0, NP - N),
                      constant_values=G).reshape(NP // 32, 32)

    x_p = jnp.pad(x, ((0, NP - N), (0, 0)))
    ha_p = jnp.pad(hedge_attr, ((0, NP - N), (0, 0)))
    zeros_np = jnp.zeros((NP, 32), f32)
    ones_e = jnp.ones((128, 32), f32)
    ones_g = jnp.ones((32, 32), f32)

    p = params
    WeT = p['embed']['W'].T
    be = p['embed']['b'][None, :]

    cnt_h = _sc_count(EP, 128, 4, 20, NP)(hidx_s, ones_e, zeros_np)
    cnt_n = _sc_count(EP, 128, 4, 20, NP)(nidx_s, ones_e, zeros_np)
    cnt_g = _sc_count(NP, 32, 7, 49, GACC)(batch_s, ones_g, zeros_np)

    h0, h1 = _t0_embed(x_p, WeT, be)

    for lp in p['layers']:
        W1 = jnp.concatenate([lp['lin_f1']['W'].T, lp['lin_c1']['W'].T], axis=1)
        b1 = jnp.concatenate([lp['lin_f1']['b'], lp['lin_c1']['b']])[None, :]
        g1 = jnp.concatenate([lp['bn_f']['g'], lp['bn_c']['g']])[None, :]
        be1 = jnp.concatenate([lp['bn_f']['b'], lp['bn_c']['b']])[None, :]
        WA = jnp.concatenate([lp['lin_f2']['W'][:, :64].T,
                              lp['lin_c2']['W'][:, :64].T], axis=1)
        bA = jnp.concatenate([lp['lin_f2']['b'], lp['lin_c2']['b']])[None, :]
        WB = jnp.concatenate([lp['lin_f2']['W'][:, 64:].T,
                              lp['lin_c2']['W'][:, 64:].T], axis=1)

        hs0 = _sc_gather_segsum(EP, 128, 4, 20, NP)(h0, nidx_g, hidx_s,
                                                    zeros_np)
        hs1 = _sc_gather_segsum(EP, 128, 4, 20, NP)(h1, nidx_g, hidx_s,
                                                    zeros_np)
        z, st = _t1a(hs0, hs1, cnt_h, ha_p, W1, b1)
        Afc, Bfc = _t1b(z, st, g1, be1, h0, h1, WA, bA, WB)
        An, Bh = _sc_gather2(EP, 128, 2, 20)(Afc, Bfc, nidx_g, hidx_g)
        m0, m1 = _t2_gate(An, Bh)
        ns0 = _sc_linear_segsum(EP, 128, 4, 20, NP)(m0, nidx_s, zeros_np)
        ns1 = _sc_linear_segsum(EP, 128, 4, 20, NP)(m1, nidx_s, zeros_np)
        nm, nst = _t3a(ns0, ns1, cnt_n)
        h0, h1 = _t3b(nm, nst, lp['bn_o']['g'][None, :],
                      lp['bn_o']['b'][None, :], h0, h1)

    gs0 = _sc_linear_segsum(NP, 32, 7, 49, GACC)(h0, batch_s, zeros_np)
    gs1 = _sc_linear_segsum(NP, 32, 7, 49, GACC)(h1, batch_s, zeros_np)

    W2 = p['l2']['W'].T
    b2 = p['l2']['b'][None, :]
    Wo = jnp.zeros((128, 128), f32).at[:, :1].set(p['out']['W'].T)
    bo = jnp.broadcast_to(p['out']['b'][None, :], (1, 128))

    out = _t4_head(gs0, gs1, cnt_g, W2, b2, Wo, bo)
    return out[:G, :1]


# speedup vs baseline: 1.1782x; 1.0003x over previous
import functools

import jax
import jax.numpy as jnp
from jax import lax
from jax.experimental import pallas as pl
from jax.experimental.pallas import tpu as pltpu
from jax.experimental.pallas import tpu_sc as plsc

N = 50000
NP = 50176
E = 800000
EP = 819200
G = 256
GACC = 272
EPS = 1e-5
NPB = 6272
EPB = 6400
f32 = jnp.float32

_mesh = lambda: plsc.VectorSubcoreMesh(core_axis_name="c", subcore_axis_name="s")
_SC_PARAMS = pltpu.CompilerParams(use_tc_tiling_on_sc=False)


@functools.lru_cache(maxsize=None)
def _sc_count(ep, k, ib, ibig, acc_rows):
    n_chunks = ep // (32 * k)
    stripe = acc_rows // 16

    @functools.partial(
        pl.kernel, mesh=_mesh(), compiler_params=_SC_PARAMS,
        out_type=jax.ShapeDtypeStruct((2, acc_rows, 32), f32),
        scratch_types=[
            pltpu.VMEM((ibig, k), jnp.int32),
            pltpu.VMEM((k, 32), f32),
            pltpu.VMEM_SHARED((acc_rows, 32), f32),
            pltpu.SemaphoreType.DMA,
        ],
    )
    def body(sidx2, ones, zeros, out, si2, ones_v, acc, sem):
        c = lax.axis_index("c")
        s = lax.axis_index("s")
        pltpu.sync_copy(zeros.at[pl.ds(s * stripe, stripe)],
                        acc.at[pl.ds(s * stripe, stripe)])
        pltpu.sync_copy(ones.at[pl.ds(0, k)], ones_v)
        plsc.subcore_barrier()
        chunk0 = (c * 16 + s) * n_chunks

        @pl.loop(0, n_chunks // ibig)
        def _(ob):
            blk0 = chunk0 + ob * ibig
            pltpu.sync_copy(sidx2.at[pl.ds(blk0, ibig)], si2)

            @pl.loop(0, ibig // ib)
            def _(g):
                cps = [pltpu.async_copy(ones_v, acc.at[si2.at[g * ib + b]],
                                        sem, add=True)
                       for b in range(ib)]
                for cp in cps:
                    cp.wait()

        plsc.subcore_barrier()
        pltpu.sync_copy(acc.at[pl.ds(s * stripe, stripe)],
                        out.at[c, pl.ds(s * stripe, stripe)])

    return body


@functools.lru_cache(maxsize=None)
def _sc_gather_segsum(ep, k, ib, ibig, acc_rows):
    n_chunks = ep // (32 * k)
    stripe = acc_rows // 16

    @functools.partial(
        pl.kernel, mesh=_mesh(), compiler_params=_SC_PARAMS,
        out_type=jax.ShapeDtypeStruct((2, acc_rows, 32), f32),
        scratch_types=[
            pltpu.VMEM((ibig, k), jnp.int32),
            pltpu.VMEM((ibig, k), jnp.int32),
            pltpu.VMEM((ib, k, 32), f32),
            pltpu.VMEM_SHARED((acc_rows, 32), f32),
            pltpu.SemaphoreType.DMA,
            pltpu.SemaphoreType.DMA,
            pltpu.SemaphoreType.DMA,
        ],
    )
    def body(table, gidx2, sidx2, zeros, out, gi2, si2, rows, acc,
             semi, semg, sems):
        c = lax.axis_index("c")
        s = lax.axis_index("s")
        pltpu.sync_copy(zeros.at[pl.ds(s * stripe, stripe)],
                        acc.at[pl.ds(s * stripe, stripe)])
        plsc.subcore_barrier()
        chunk0 = (c * 16 + s) * n_chunks
        n_inner = ibig // ib

        @pl.loop(0, n_chunks // ibig)
        def _(ob):
            @pl.when(ob > 0)
            def _():
                for b in range(ib):
                    pltpu.make_async_copy(
                        zeros.at[pl.ds(0, k)], rows.at[b], sems).wait()

            blk0 = chunk0 + ob * ibig
            cg = pltpu.async_copy(gidx2.at[pl.ds(blk0, ibig)], gi2, semi)
            cs = pltpu.async_copy(sidx2.at[pl.ds(blk0, ibig)], si2, semi)
            cg.wait()
            cs.wait()

            @pl.loop(0, n_inner)
            def _(g):
                @pl.when(g > 0)
                def _():
                    for b in range(ib):
                        pltpu.make_async_copy(
                            zeros.at[pl.ds(0, k)], rows.at[b], sems).wait()

                gs = [pltpu.async_copy(table.at[gi2.at[g * ib + b]],
                                       rows.at[b], semg)
                      for b in range(ib)]
                for b in range(ib):
                    gs[b].wait()
                    pltpu.async_copy(rows.at[b], acc.at[si2.at[g * ib + b]],
                                     sems, add=True)

        for b in range(ib):
            pltpu.make_async_copy(
                zeros.at[pl.ds(0, k)], rows.at[b], sems).wait()
        plsc.subcore_barrier()
        pltpu.sync_copy(acc.at[pl.ds(s * stripe, stripe)],
                        out.at[c, pl.ds(s * stripe, stripe)])

    return body


@functools.lru_cache(maxsize=None)
def _sc_linear_segsum(ep, k, ib, ibig, acc_rows):
    n_chunks = ep // (32 * k)
    stripe = acc_rows // 16

    @functools.partial(
        pl.kernel, mesh=_mesh(), compiler_params=_SC_PARAMS,
        out_type=jax.ShapeDtypeStruct((2, acc_rows, 32), f32),
        scratch_types=[
            pltpu.VMEM((ibig, k), jnp.int32),
            pltpu.VMEM((ib * k, 32), f32),
            pltpu.VMEM_SHARED((acc_rows, 32), f32),
            pltpu.SemaphoreType.DMA,
        ],
    )
    def body(rows2d, sidx2, zeros, out, si2, rows_v, acc, sem):
        c = lax.axis_index("c")
        s = lax.axis_index("s")
        pltpu.sync_copy(zeros.at[pl.ds(s * stripe, stripe)],
                        acc.at[pl.ds(s * stripe, stripe)])
        plsc.subcore_barrier()
        chunk0 = (c * 16 + s) * n_chunks

        @pl.loop(0, n_chunks // ibig)
        def _(ob):
            @pl.when(ob > 0)
            def _():
                for b in range(ib):
                    pltpu.make_async_copy(
                        zeros.at[pl.ds(0, k)],
                        rows_v.at[pl.ds(b * k, k)], sem).wait()

            blk0 = chunk0 + ob * ibig
            pltpu.sync_copy(sidx2.at[pl.ds(blk0, ibig)], si2)

            @pl.loop(0, ibig // ib)
            def _(g):
                @pl.when(g > 0)
                def _():
                    for b in range(ib):
                        pltpu.make_async_copy(
                            zeros.at[pl.ds(0, k)],
                            rows_v.at[pl.ds(b * k, k)], sem).wait()

                blk = blk0 + g * ib
                pltpu.sync_copy(rows2d.at[pl.ds(blk * k, ib * k)], rows_v)
                for b in range(ib):
                    pltpu.async_copy(rows_v.at[pl.ds(b * k, k)],
                                     acc.at[si2.at[g * ib + b]],
                                     sem, add=True)

        for b in range(ib):
            pltpu.make_async_copy(
                zeros.at[pl.ds(0, k)], rows_v.at[pl.ds(b * k, k)], sem).wait()
        plsc.subcore_barrier()
        pltpu.sync_copy(acc.at[pl.ds(s * stripe, stripe)],
                        out.at[c, pl.ds(s * stripe, stripe)])

    return body


@functools.lru_cache(maxsize=None)
def _sc_gather2(ep, k, ib, ibig):
    n_chunks = ep // (32 * k)

    @functools.partial(
        pl.kernel, mesh=_mesh(), compiler_params=_SC_PARAMS,
        out_type=(jax.ShapeDtypeStruct((ep, 128), f32),
                  jax.ShapeDtypeStruct((ep, 128), f32)),
        scratch_types=[
            pltpu.VMEM((ibig, k), jnp.int32),
            pltpu.VMEM((ibig, k), jnp.int32),
            pltpu.VMEM((ib, k, 128), f32),
            pltpu.VMEM((ib, k, 128), f32),
            pltpu.SemaphoreType.DMA,
            pltpu.SemaphoreType.DMA,
            pltpu.SemaphoreType.DMA,
        ],
    )
    def body(tA, tB, idxA2, idxB2, outA, outB, ia2, ib2, bufA, bufB,
             semi, semg, semw):
        c = lax.axis_index("c")
        s = lax.axis_index("s")
        chunk0 = (c * 16 + s) * n_chunks

        def drain_writes():
            for b in range(ib):
                pltpu.make_async_copy(
                    tA.at[pl.ds(0, k)], bufA.at[b], semw).wait()
                pltpu.make_async_copy(
                    tA.at[pl.ds(0, k)], bufB.at[b], semw).wait()

        @pl.loop(0, n_chunks // ibig)
        def _(ob):
            @pl.when(ob > 0)
            def _():
                drain_writes()

            blk0 = chunk0 + ob * ibig
            ca = pltpu.async_copy(idxA2.at[pl.ds(blk0, ibig)], ia2, semi)
            cb = pltpu.async_copy(idxB2.at[pl.ds(blk0, ibig)], ib2, semi)
            ca.wait()
            cb.wait()

            @pl.loop(0, ibig // ib)
            def _(g):
                @pl.when(g > 0)
                def _():
                    drain_writes()

                blk = blk0 + g * ib
                gs = []
                for b in range(ib):
                    gs.append(pltpu.async_copy(
                        tA.at[ia2.at[g * ib + b]], bufA.at[b], semg))
                    gs.append(pltpu.async_copy(
                        tB.at[ib2.at[g * ib + b]], bufB.at[b], semg))
                for b in range(ib):
                    gs[2 * b].wait()
                    pltpu.async_copy(
                        bufA.at[b], outA.at[pl.ds((blk + b) * k, k)], semw)
                    gs[2 * b + 1].wait()
                    pltpu.async_copy(
                        bufB.at[b], outB.at[pl.ds((blk + b) * k, k)], semw)

        drain_writes()

    return body


def _t0_embed(x_p, WeT, be):
    def body(x_ref, w_ref, b_ref, h0_ref, h1_ref):
        h = jnp.dot(x_ref[...], w_ref[...], preferred_element_type=f32) + b_ref[...]
        h0_ref[...] = h[:, :32]
        h1_ref[...] = h[:, 32:]

    return pl.pallas_call(
        body,
        grid=(NP // NPB,),
        in_specs=[
            pl.BlockSpec((NPB, 92), lambda i: (i, 0)),
            pl.BlockSpec((92, 64), lambda i: (0, 0)),
            pl.BlockSpec((1, 64), lambda i: (0, 0)),
        ],
        out_specs=[pl.BlockSpec((NPB, 32), lambda i: (i, 0))] * 2,
        out_shape=[jax.ShapeDtypeStruct((NP, 32), f32)] * 2,
    )(x_p, WeT, be)


def _t1a(hs0, hs1, cnt_h, ha_p, W1, b1):
    nb = NP // NPB

    def body(hs0_ref, hs1_ref, cnt_ref, ha_ref, w_ref, b_ref,
             z_ref, st_ref, acc):
        i = pl.program_id(0)
        c = cnt_ref[0, :, :1] + cnt_ref[1, :, :1]
        r = 1.0 / jnp.maximum(c, 1.0)
        hm0 = (hs0_ref[0] + hs0_ref[1]) * r
        hm1 = (hs1_ref[0] + hs1_ref[1]) * r
        msg = jnp.concatenate([hm0, hm1, ha_ref[...]], axis=1)
        z = jnp.dot(msg, w_ref[...], preferred_element_type=f32) + b_ref[...]
        z_ref[...] = z
        rows = i * NPB + lax.broadcasted_iota(jnp.int32, (NPB, 1), 0)
        zm = jnp.where(rows < N, z, 0.0)
        s1 = jnp.sum(zm, axis=0)
        s2 = jnp.sum(zm * zm, axis=0)
        upd = jnp.concatenate(
            [s1[None, :], s2[None, :], jnp.zeros((6, 70), f32)], axis=0)

        @pl.when(i == 0)
        def _():
            acc[...] = jnp.zeros_like(acc)

        acc[...] += upd

        @pl.when(i == nb - 1)
        def _():
            st_ref[...] = acc[...]

    return pl.pallas_call(
        body,
        grid=(nb,),
        in_specs=[
            pl.BlockSpec((2, NPB, 32), lambda i: (0, i, 0)),
            pl.BlockSpec((2, NPB, 32), lambda i: (0, i, 0)),
            pl.BlockSpec((2, NPB, 32), lambda i: (0, i, 0)),
            pl.BlockSpec((NPB, 35), lambda i: (i, 0)),
            pl.BlockSpec((99, 70), lambda i: (0, 0)),
            pl.BlockSpec((1, 70), lambda i: (0, 0)),
        ],
        out_specs=[
            pl.BlockSpec((NPB, 70), lambda i: (i, 0)),
            pl.BlockSpec((8, 70), lambda i: (0, 0)),
        ],
        out_shape=[
            jax.ShapeDtypeStruct((NP, 70), f32),
            jax.ShapeDtypeStruct((8, 70), f32),
        ],
        scratch_shapes=[pltpu.VMEM((8, 70), f32)],
    )(hs0, hs1, cnt_h, ha_p, W1, b1)


def _t1b(z, st, g1, be1, h0, h1, WA, bA, WB):
    def body(z_ref, st_ref, g_ref, be_ref, h0_ref, h1_ref,
             wa_ref, ba_ref, wb_ref, afc_ref, bfc_ref):
        mean = st_ref[0, :] * (1.0 / N)
        var = st_ref[1, :] * (1.0 / N) - mean * mean
        scale = g_ref[0, :] * lax.rsqrt(var + EPS)
        zn = (z_ref[...] - mean[None, :]) * scale[None, :] + be_ref[...]
        ha = jax.nn.sigmoid(zn[:, :35]) * jax.nn.softplus(zn[:, 35:70])
        bfc_ref[...] = jnp.dot(ha, wb_ref[...], preferred_element_type=f32)
        h = jnp.concatenate([h0_ref[...], h1_ref[...]], axis=1)
        afc_ref[...] = jnp.dot(h, wa_ref[...],
                               preferred_element_type=f32) + ba_ref[...]

    return pl.pallas_call(
        body,
        grid=(NP // NPB,),
        in_specs=[
            pl.BlockSpec((NPB, 70), lambda i: (i, 0)),
            pl.BlockSpec((8, 70), lambda i: (0, 0)),
            pl.BlockSpec((1, 70), lambda i: (0, 0)),
            pl.BlockSpec((1, 70), lambda i: (0, 0)),
            pl.BlockSpec((NPB, 32), lambda i: (i, 0)),
            pl.BlockSpec((NPB, 32), lambda i: (i, 0)),
            pl.BlockSpec((64, 128), lambda i: (0, 0)),
            pl.BlockSpec((1, 128), lambda i: (0, 0)),
            pl.BlockSpec((35, 128), lambda i: (0, 0)),
        ],
        out_specs=[
            pl.BlockSpec((NPB, 128), lambda i: (i, 0)),
            pl.BlockSpec((NPB, 128), lambda i: (i, 0)),
        ],
        out_shape=[jax.ShapeDtypeStruct((NP, 128), f32)] * 2,
    )(z, st, g1, be1, h0, h1, WA, bA, WB)


def _t2_gate(An, Bh):
    def body(a_ref, b_ref, m0_ref, m1_ref):
        e = a_ref[...] + b_ref[...]
        m = jax.nn.sigmoid(e[:, :64]) * jax.nn.softplus(e[:, 64:])
        m0_ref[...] = m[:, :32]
        m1_ref[...] = m[:, 32:]

    return pl.pallas_call(
        body,
        grid=(EP // EPB,),
        in_specs=[
            pl.BlockSpec((EPB, 128), lambda i: (i, 0)),
            pl.BlockSpec((EPB, 128), lambda i: (i, 0)),
        ],
        out_specs=[pl.BlockSpec((EPB, 32), lambda i: (i, 0))] * 2,
        out_shape=[jax.ShapeDtypeStruct((EP, 32), f32)] * 2,
    )(An, Bh)


def _t3a(ns0, ns1, cnt_n):
    nb = NP // NPB

    def body(ns0_ref, ns1_ref, cnt_ref, nm_ref, st_ref, acc):
        i = pl.program_id(0)
        c = cnt_ref[0, :, :1] + cnt_ref[1, :, :1]
        r = 1.0 / jnp.maximum(c, 1.0)
        nm = jnp.concatenate([(ns0_ref[0] + ns0_ref[1]) * r,
                              (ns1_ref[0] + ns1_ref[1]) * r], axis=1)
        nm_ref[...] = nm
        rows = i * NPB + lax.broadcasted_iota(jnp.int32, (NPB, 1), 0)
        nmm = jnp.where(rows < N, nm, 0.0)
        s1 = jnp.sum(nmm, axis=0)
        s2 = jnp.sum(nmm * nmm, axis=0)
        upd = jnp.concatenate(
            [s1[None, :], s2[None, :], jnp.zeros((6, 64), f32)], axis=0)

        @pl.when(i == 0)
        def _():
            acc[...] = jnp.zeros_like(acc)

        acc[...] += upd

        @pl.when(i == nb - 1)
        def _():
            st_ref[...] = acc[...]

    return pl.pallas_call(
        body,
        grid=(nb,),
        in_specs=[
            pl.BlockSpec((2, NPB, 32), lambda i: (0, i, 0)),
            pl.BlockSpec((2, NPB, 32), lambda i: (0, i, 0)),
            pl.BlockSpec((2, NPB, 32), lambda i: (0, i, 0)),
        ],
        out_specs=[
            pl.BlockSpec((NPB, 64), lambda i: (i, 0)),
            pl.BlockSpec((8, 64), lambda i: (0, 0)),
        ],
        out_shape=[
            jax.ShapeDtypeStruct((NP, 64), f32),
            jax.ShapeDtypeStruct((8, 64), f32),
        ],
        scratch_shapes=[pltpu.VMEM((8, 64), f32)],
    )(ns0, ns1, cnt_n)


def _t3b(nm, st, go, bo, h0, h1):
    def body(nm_ref, st_ref, g_ref, b_ref, h0_ref, h1_ref, o0_ref, o1_ref):
        mean = st_ref[0, :] * (1.0 / N)
        var = st_ref[1, :] * (1.0 / N) - mean * mean
        scale = g_ref[0, :] * lax.rsqrt(var + EPS)
        y = (nm_ref[...] - mean[None, :]) * scale[None, :] + b_ref[...]
        h = jnp.concatenate([h0_ref[...], h1_ref[...]], axis=1)
        hn = jax.nn.relu(jax.nn.softplus(y + h))
        o0_ref[...] = hn[:, :32]
        o1_ref[...] = hn[:, 32:]

    return pl.pallas_call(
        body,
        grid=(NP // NPB,),
        in_specs=[
            pl.BlockSpec((NPB, 64), lambda i: (i, 0)),
            pl.BlockSpec((8, 64), lambda i: (0, 0)),
            pl.BlockSpec((1, 64), lambda i: (0, 0)),
            pl.BlockSpec((1, 64), lambda i: (0, 0)),
            pl.BlockSpec((NPB, 32), lambda i: (i, 0)),
            pl.BlockSpec((NPB, 32), lambda i: (i, 0)),
        ],
        out_specs=[pl.BlockSpec((NPB, 32), lambda i: (i, 0))] * 2,
        out_shape=[jax.ShapeDtypeStruct((NP, 32), f32)] * 2,
    )(nm, st, go, bo, h0, h1)


def _t4_head(gs0, gs1, cnt_g, W2, b2, Wo, bo):
    def body(gs0_ref, gs1_ref, cnt_ref, w2_ref, b2_ref, wo_ref, bo_ref, o_ref):
        c = cnt_ref[0, :, :1] + cnt_ref[1, :, :1]
        r = 1.0 / jnp.maximum(c, 1.0)
        g = jnp.concatenate([(gs0_ref[0] + gs0_ref[1]) * r,
                             (gs1_ref[0] + gs1_ref[1]) * r], axis=1)
        t = jax.nn.softplus(
            jnp.dot(g, w2_ref[...], preferred_element_type=f32) + b2_ref[...])
        o_ref[...] = jnp.dot(t, wo_ref[...],
                             preferred_element_type=f32) + bo_ref[...]

    return pl.pallas_call(
        body,
        in_specs=[
            pl.BlockSpec((2, GACC, 32), lambda: (0, 0, 0)),
            pl.BlockSpec((2, GACC, 32), lambda: (0, 0, 0)),
            pl.BlockSpec((2, GACC, 32), lambda: (0, 0, 0)),
            pl.BlockSpec((64, 128), lambda: (0, 0)),
            pl.BlockSpec((1, 128), lambda: (0, 0)),
            pl.BlockSpec((128, 128), lambda: (0, 0)),
            pl.BlockSpec((1, 128), lambda: (0, 0)),
        ],
        out_specs=pl.BlockSpec((GACC, 128), lambda: (0, 0)),
        out_shape=jax.ShapeDtypeStruct((GACC, 128), f32),
    )(gs0, gs1, cnt_g, W2, b2, Wo, bo)


def kernel(x, hyperedge_index, hedge_attr, batch, params):
    nidx = hyperedge_index[0]
    hidx = hyperedge_index[1]
    nidx_g = jnp.pad(nidx, (0, EP - E)).reshape(EP // 128, 128)
    nidx_s = jnp.pad(nidx, (0, EP - E),
                     constant_values=N).reshape(EP // 128, 128)
    hidx_g = jnp.pad(hidx, (0, EP - E)).reshape(EP // 128, 128)
    hidx_s = jnp.pad(hidx, (0, EP - E),
                     constant_values=N).reshape(EP // 128, 128)
    batch_s = jnp.pad(batch, (0, NP - N),
                      constant_values=G).reshape(NP // 32, 32)

    x_p = jnp.pad(x, ((0, NP - N), (0, 0)))
    ha_p = jnp.pad(hedge_attr, ((0, NP - N), (0, 0)))
    zeros_np = jnp.zeros((NP, 32), f32)
    ones_e = jnp.ones((128, 32), f32)
    ones_g = jnp.ones((32, 32), f32)

    p = params
    WeT = p['embed']['W'].T
    be = p['embed']['b'][None, :]

    cnt_h = _sc_count(EP, 128, 4, 20, NP)(hidx_s, ones_e, zeros_np)
    cnt_n = _sc_count(EP, 128, 4, 20, NP)(nidx_s, ones_e, zeros_np)
    cnt_g = _sc_count(NP, 32, 7, 49, GACC)(batch_s, ones_g, zeros_np)

    h0, h1 = _t0_embed(x_p, WeT, be)

    for lp in p['layers']:
        W1 = jnp.concatenate([lp['lin_f1']['W'].T, lp['lin_c1']['W'].T], axis=1)
        b1 = jnp.concatenate([lp['lin_f1']['b'], lp['lin_c1']['b']])[None, :]
        g1 = jnp.concatenate([lp['bn_f']['g'], lp['bn_c']['g']])[None, :]
        be1 = jnp.concatenate([lp['bn_f']['b'], lp['bn_c']['b']])[None, :]
        WA = jnp.concatenate([lp['lin_f2']['W'][:, :64].T,
                              lp['lin_c2']['W'][:, :64].T], axis=1)
        bA = jnp.concatenate([lp['lin_f2']['b'], lp['lin_c2']['b']])[None, :]
        WB = jnp.concatenate([lp['lin_f2']['W'][:, 64:].T,
                              lp['lin_c2']['W'][:, 64:].T], axis=1)

        hs0 = _sc_gather_segsum(EP, 128, 4, 20, NP)(h0, nidx_g, hidx_s,
                                                    zeros_np)
        hs1 = _sc_gather_segsum(EP, 128, 4, 20, NP)(h1, nidx_g, hidx_s,
                                                    zeros_np)
        z, st = _t1a(hs0, hs1, cnt_h, ha_p, W1, b1)
        Afc, Bfc = _t1b(z, st, g1, be1, h0, h1, WA, bA, WB)
        An, Bh = _sc_gather2(EP, 128, 2, 20)(Afc, Bfc, nidx_g, hidx_g)
        m0, m1 = _t2_gate(An, Bh)
        ns0 = _sc_linear_segsum(EP, 128, 4, 20, NP)(m0, nidx_s, zeros_np)
        ns1 = _sc_linear_segsum(EP, 128, 4, 20, NP)(m1, nidx_s, zeros_np)
        nm, nst = _t3a(ns0, ns1, cnt_n)
        h0, h1 = _t3b(nm, nst, lp['bn_o']['g'][None, :],
                      lp['bn_o']['b'][None, :], h0, h1)

    gs0 = _sc_linear_segsum(NP, 32, 7, 49, GACC)(h0, batch_s, zeros_np)
    gs1 = _sc_linear_segsum(NP, 32, 7, 49, GACC)(h1, batch_s, zeros_np)

    W2 = p['l2']['W'].T
    b2 = p['l2']['b'][None, :]
    Wo = jnp.zeros((128, 128), f32).at[:, :1].set(p['out']['W'].T)
    bo = jnp.broadcast_to(p['out']['b'][None, :], (1, 128))

    out = _t4_head(gs0, gs1, cnt_g, W2, b2, Wo, bo)
    return out[:G, :1]


# speedup vs baseline: 1.2084x; 1.0256x over previous
import functools

import jax
import jax.numpy as jnp
from jax import lax
from jax.experimental import pallas as pl
from jax.experimental.pallas import tpu as pltpu
from jax.experimental.pallas import tpu_sc as plsc

N = 50000
NP = 50176
E = 800000
EP = 819200
G = 256
GACC = 272
EPS = 1e-5
NPB = 6272
EPB = 3200
f32 = jnp.float32

_mesh = lambda: plsc.VectorSubcoreMesh(core_axis_name="c", subcore_axis_name="s")
_SC_PARAMS = pltpu.CompilerParams(use_tc_tiling_on_sc=False)


@functools.lru_cache(maxsize=None)
def _sc_count(ep, k, ib, ibig, acc_rows):
    n_chunks = ep // (32 * k)
    stripe = acc_rows // 16

    @functools.partial(
        pl.kernel, mesh=_mesh(), compiler_params=_SC_PARAMS,
        out_type=jax.ShapeDtypeStruct((2, acc_rows, 32), f32),
        scratch_types=[
            pltpu.VMEM((ibig, k), jnp.int32),
            pltpu.VMEM((k, 32), f32),
            pltpu.VMEM_SHARED((acc_rows, 32), f32),
            pltpu.SemaphoreType.DMA,
        ],
    )
    def body(sidx2, ones, zeros, out, si2, ones_v, acc, sem):
        c = lax.axis_index("c")
        s = lax.axis_index("s")
        pltpu.sync_copy(zeros.at[pl.ds(s * stripe, stripe)],
                        acc.at[pl.ds(s * stripe, stripe)])
        pltpu.sync_copy(ones.at[pl.ds(0, k)], ones_v)
        plsc.subcore_barrier()
        chunk0 = (c * 16 + s) * n_chunks

        @pl.loop(0, n_chunks // ibig)
        def _(ob):
            blk0 = chunk0 + ob * ibig
            pltpu.sync_copy(sidx2.at[pl.ds(blk0, ibig)], si2)

            @pl.loop(0, ibig // ib)
            def _(g):
                cps = [pltpu.async_copy(ones_v, acc.at[si2.at[g * ib + b]],
                                        sem, add=True)
                       for b in range(ib)]
                for cp in cps:
                    cp.wait()

        plsc.subcore_barrier()
        pltpu.sync_copy(acc.at[pl.ds(s * stripe, stripe)],
                        out.at[c, pl.ds(s * stripe, stripe)])

    return body


@functools.lru_cache(maxsize=None)
def _sc_gather_segsum(ep, k, ib, ibig, acc_rows):
    n_chunks = ep // (32 * k)
    stripe = acc_rows // 16

    @functools.partial(
        pl.kernel, mesh=_mesh(), compiler_params=_SC_PARAMS,
        out_type=jax.ShapeDtypeStruct((2, acc_rows, 32), f32),
        scratch_types=[
            pltpu.VMEM((ibig, k), jnp.int32),
            pltpu.VMEM((ibig, k), jnp.int32),
            pltpu.VMEM((ib, k, 32), f32),
            pltpu.VMEM_SHARED((acc_rows, 32), f32),
            pltpu.SemaphoreType.DMA,
            pltpu.SemaphoreType.DMA,
            pltpu.SemaphoreType.DMA,
        ],
    )
    def body(table, gidx2, sidx2, zeros, out, gi2, si2, rows, acc,
             semi, semg, sems):
        c = lax.axis_index("c")
        s = lax.axis_index("s")
        pltpu.sync_copy(zeros.at[pl.ds(s * stripe, stripe)],
                        acc.at[pl.ds(s * stripe, stripe)])
        plsc.subcore_barrier()
        chunk0 = (c * 16 + s) * n_chunks
        n_inner = ibig // ib

        @pl.loop(0, n_chunks // ibig)
        def _(ob):
            @pl.when(ob > 0)
            def _():
                for b in range(ib):
                    pltpu.make_async_copy(
                        zeros.at[pl.ds(0, k)], rows.at[b], sems).wait()

            blk0 = chunk0 + ob * ibig
            cg = pltpu.async_copy(gidx2.at[pl.ds(blk0, ibig)], gi2, semi)
            cs = pltpu.async_copy(sidx2.at[pl.ds(blk0, ibig)], si2, semi)
            cg.wait()
            cs.wait()

            @pl.loop(0, n_inner)
            def _(g):
                @pl.when(g > 0)
                def _():
                    for b in range(ib):
                        pltpu.make_async_copy(
                            zeros.at[pl.ds(0, k)], rows.at[b], sems).wait()

                gs = [pltpu.async_copy(table.at[gi2.at[g * ib + b]],
                                       rows.at[b], semg)
                      for b in range(ib)]
                for b in range(ib):
                    gs[b].wait()
                    pltpu.async_copy(rows.at[b], acc.at[si2.at[g * ib + b]],
                                     sems, add=True)

        for b in range(ib):
            pltpu.make_async_copy(
                zeros.at[pl.ds(0, k)], rows.at[b], sems).wait()
        plsc.subcore_barrier()
        pltpu.sync_copy(acc.at[pl.ds(s * stripe, stripe)],
                        out.at[c, pl.ds(s * stripe, stripe)])

    return body


@functools.lru_cache(maxsize=None)
def _sc_linear_segsum(ep, k, ib, ibig, acc_rows):
    n_chunks = ep // (32 * k)
    stripe = acc_rows // 16

    @functools.partial(
        pl.kernel, mesh=_mesh(), compiler_params=_SC_PARAMS,
        out_type=jax.ShapeDtypeStruct((2, acc_rows, 32), f32),
        scratch_types=[
            pltpu.VMEM((ibig, k), jnp.int32),
            pltpu.VMEM((ib * k, 32), f32),
            pltpu.VMEM_SHARED((acc_rows, 32), f32),
            pltpu.SemaphoreType.DMA,
        ],
    )
    def body(rows2d, sidx2, zeros, out, si2, rows_v, acc, sem):
        c = lax.axis_index("c")
        s = lax.axis_index("s")
        pltpu.sync_copy(zeros.at[pl.ds(s * stripe, stripe)],
                        acc.at[pl.ds(s * stripe, stripe)])
        plsc.subcore_barrier()
        chunk0 = (c * 16 + s) * n_chunks

        @pl.loop(0, n_chunks // ibig)
        def _(ob):
            @pl.when(ob > 0)
            def _():
                for b in range(ib):
                    pltpu.make_async_copy(
                        zeros.at[pl.ds(0, k)],
                        rows_v.at[pl.ds(b * k, k)], sem).wait()

            blk0 = chunk0 + ob * ibig
            pltpu.sync_copy(sidx2.at[pl.ds(blk0, ibig)], si2)

            @pl.loop(0, ibig // ib)
            def _(g):
                @pl.when(g > 0)
                def _():
                    for b in range(ib):
                        pltpu.make_async_copy(
                            zeros.at[pl.ds(0, k)],
                            rows_v.at[pl.ds(b * k, k)], sem).wait()

                blk = blk0 + g * ib
                pltpu.sync_copy(rows2d.at[pl.ds(blk * k, ib * k)], rows_v)
                for b in range(ib):
                    pltpu.async_copy(rows_v.at[pl.ds(b * k, k)],
                                     acc.at[si2.at[g * ib + b]],
                                     sem, add=True)

        for b in range(ib):
            pltpu.make_async_copy(
                zeros.at[pl.ds(0, k)], rows_v.at[pl.ds(b * k, k)], sem).wait()
        plsc.subcore_barrier()
        pltpu.sync_copy(acc.at[pl.ds(s * stripe, stripe)],
                        out.at[c, pl.ds(s * stripe, stripe)])

    return body


@functools.lru_cache(maxsize=None)
def _sc_gather2(ep, k, ib, ibig):
    n_chunks = ep // (32 * k)

    @functools.partial(
        pl.kernel, mesh=_mesh(), compiler_params=_SC_PARAMS,
        out_type=(jax.ShapeDtypeStruct((ep, 128), f32),
                  jax.ShapeDtypeStruct((ep, 128), f32)),
        scratch_types=[
            pltpu.VMEM((ibig, k), jnp.int32),
            pltpu.VMEM((ibig, k), jnp.int32),
            pltpu.VMEM((ib, k, 128), f32),
            pltpu.VMEM((ib, k, 128), f32),
            pltpu.SemaphoreType.DMA,
            pltpu.SemaphoreType.DMA,
            pltpu.SemaphoreType.DMA,
        ],
    )
    def body(tA, tB, idxA2, idxB2, outA, outB, ia2, ib2, bufA, bufB,
             semi, semg, semw):
        c = lax.axis_index("c")
        s = lax.axis_index("s")
        chunk0 = (c * 16 + s) * n_chunks

        def drain_writes():
            for b in range(ib):
                pltpu.make_async_copy(
                    tA.at[pl.ds(0, k)], bufA.at[b], semw).wait()
                pltpu.make_async_copy(
                    tA.at[pl.ds(0, k)], bufB.at[b], semw).wait()

        @pl.loop(0, n_chunks // ibig)
        def _(ob):
            @pl.when(ob > 0)
            def _():
                drain_writes()

            blk0 = chunk0 + ob * ibig
            ca = pltpu.async_copy(idxA2.at[pl.ds(blk0, ibig)], ia2, semi)
            cb = pltpu.async_copy(idxB2.at[pl.ds(blk0, ibig)], ib2, semi)
            ca.wait()
            cb.wait()

            @pl.loop(0, ibig // ib)
            def _(g):
                @pl.when(g > 0)
                def _():
                    drain_writes()

                blk = blk0 + g * ib
                gs = []
                for b in range(ib):
                    gs.append(pltpu.async_copy(
                        tA.at[ia2.at[g * ib + b]], bufA.at[b], semg))
                    gs.append(pltpu.async_copy(
                        tB.at[ib2.at[g * ib + b]], bufB.at[b], semg))
                for b in range(ib):
                    gs[2 * b].wait()
                    pltpu.async_copy(
                        bufA.at[b], outA.at[pl.ds((blk + b) * k, k)], semw)
                    gs[2 * b + 1].wait()
                    pltpu.async_copy(
                        bufB.at[b], outB.at[pl.ds((blk + b) * k, k)], semw)

        drain_writes()

    return body


def _t0_embed(x_p, WeT, be):
    def body(x_ref, w_ref, b_ref, h0_ref, h1_ref):
        h = jnp.dot(x_ref[...], w_ref[...], preferred_element_type=f32) + b_ref[...]
        h0_ref[...] = h[:, :32]
        h1_ref[...] = h[:, 32:]

    return pl.pallas_call(
        body,
        grid=(NP // NPB,),
        in_specs=[
            pl.BlockSpec((NPB, 92), lambda i: (i, 0)),
            pl.BlockSpec((92, 64), lambda i: (0, 0)),
            pl.BlockSpec((1, 64), lambda i: (0, 0)),
        ],
        out_specs=[pl.BlockSpec((NPB, 32), lambda i: (i, 0))] * 2,
        out_shape=[jax.ShapeDtypeStruct((NP, 32), f32)] * 2,
    )(x_p, WeT, be)


def _t1a(hs0, hs1, cnt_h, ha_p, W1, b1):
    nb = NP // NPB

    def body(hs0_ref, hs1_ref, cnt_ref, ha_ref, w_ref, b_ref,
             z_ref, st_ref, acc):
        i = pl.program_id(0)
        c = cnt_ref[0, :, :1] + cnt_ref[1, :, :1]
        r = 1.0 / jnp.maximum(c, 1.0)
        hm0 = (hs0_ref[0] + hs0_ref[1]) * r
        hm1 = (hs1_ref[0] + hs1_ref[1]) * r
        msg = jnp.concatenate([hm0, hm1, ha_ref[...]], axis=1)
        z = jnp.dot(msg, w_ref[...], preferred_element_type=f32) + b_ref[...]
        z_ref[...] = z
        rows = i * NPB + lax.broadcasted_iota(jnp.int32, (NPB, 1), 0)
        zm = jnp.where(rows < N, z, 0.0)
        s1 = jnp.sum(zm, axis=0)
        s2 = jnp.sum(zm * zm, axis=0)
        upd = jnp.concatenate(
            [s1[None, :], s2[None, :], jnp.zeros((6, 70), f32)], axis=0)

        @pl.when(i == 0)
        def _():
            acc[...] = jnp.zeros_like(acc)

        acc[...] += upd

        @pl.when(i == nb - 1)
        def _():
            st_ref[...] = acc[...]

    return pl.pallas_call(
        body,
        grid=(nb,),
        in_specs=[
            pl.BlockSpec((2, NPB, 32), lambda i: (0, i, 0)),
            pl.BlockSpec((2, NPB, 32), lambda i: (0, i, 0)),
            pl.BlockSpec((2, NPB, 32), lambda i: (0, i, 0)),
            pl.BlockSpec((NPB, 35), lambda i: (i, 0)),
            pl.BlockSpec((99, 70), lambda i: (0, 0)),
            pl.BlockSpec((1, 70), lambda i: (0, 0)),
        ],
        out_specs=[
            pl.BlockSpec((NPB, 70), lambda i: (i, 0)),
            pl.BlockSpec((8, 70), lambda i: (0, 0)),
        ],
        out_shape=[
            jax.ShapeDtypeStruct((NP, 70), f32),
            jax.ShapeDtypeStruct((8, 70), f32),
        ],
        scratch_shapes=[pltpu.VMEM((8, 70), f32)],
    )(hs0, hs1, cnt_h, ha_p, W1, b1)


def _t1b(z, st, g1, be1, h0, h1, WA, bA, WB):
    def body(z_ref, st_ref, g_ref, be_ref, h0_ref, h1_ref,
             wa_ref, ba_ref, wb_ref, afc_ref, bfc_ref):
        mean = st_ref[0, :] * (1.0 / N)
        var = st_ref[1, :] * (1.0 / N) - mean * mean
        scale = g_ref[0, :] * lax.rsqrt(var + EPS)
        zn = (z_ref[...] - mean[None, :]) * scale[None, :] + be_ref[...]
        ha = jax.nn.sigmoid(zn[:, :35]) * jax.nn.softplus(zn[:, 35:70])
        bfc_ref[...] = jnp.dot(ha, wb_ref[...], preferred_element_type=f32)
        h = jnp.concatenate([h0_ref[...], h1_ref[...]], axis=1)
        afc_ref[...] = jnp.dot(h, wa_ref[...],
                               preferred_element_type=f32) + ba_ref[...]

    return pl.pallas_call(
        body,
        grid=(NP // NPB,),
        in_specs=[
            pl.BlockSpec((NPB, 70), lambda i: (i, 0)),
            pl.BlockSpec((8, 70), lambda i: (0, 0)),
            pl.BlockSpec((1, 70), lambda i: (0, 0)),
            pl.BlockSpec((1, 70), lambda i: (0, 0)),
            pl.BlockSpec((NPB, 32), lambda i: (i, 0)),
            pl.BlockSpec((NPB, 32), lambda i: (i, 0)),
            pl.BlockSpec((64, 128), lambda i: (0, 0)),
            pl.BlockSpec((1, 128), lambda i: (0, 0)),
            pl.BlockSpec((35, 128), lambda i: (0, 0)),
        ],
        out_specs=[
            pl.BlockSpec((NPB, 128), lambda i: (i, 0)),
            pl.BlockSpec((NPB, 128), lambda i: (i, 0)),
        ],
        out_shape=[jax.ShapeDtypeStruct((NP, 128), f32)] * 2,
    )(z, st, g1, be1, h0, h1, WA, bA, WB)


def _t2_gate(An, Bh):
    rows = An.shape[0]

    def body(a_ref, b_ref, m0_ref, m1_ref):
        e = a_ref[...] + b_ref[...]
        m = jax.nn.sigmoid(e[:, :64]) * jax.nn.softplus(e[:, 64:])
        m0_ref[...] = m[:, :32]
        m1_ref[...] = m[:, 32:]

    return pl.pallas_call(
        body,
        grid=(rows // EPB,),
        in_specs=[
            pl.BlockSpec((EPB, 128), lambda i: (i, 0)),
            pl.BlockSpec((EPB, 128), lambda i: (i, 0)),
        ],
        out_specs=[pl.BlockSpec((EPB, 32), lambda i: (i, 0))] * 2,
        out_shape=[jax.ShapeDtypeStruct((rows, 32), f32)] * 2,
    )(An, Bh)


def _t_sum2(a, b):
    av = a.reshape(2, NP // 4, 128)
    bv = b.reshape(2, NP // 4, 128)
    rb = NP // 4 // 8

    def body(a_ref, b_ref, o_ref):
        o_ref[...] = a_ref[...] + b_ref[...]

    out = pl.pallas_call(
        body,
        grid=(8,),
        in_specs=[pl.BlockSpec((2, rb, 128), lambda i: (0, i, 0))] * 2,
        out_specs=pl.BlockSpec((2, rb, 128), lambda i: (0, i, 0)),
        out_shape=jax.ShapeDtypeStruct((2, NP // 4, 128), f32),
    )(av, bv)
    return out.reshape(2, NP, 32)


def _t3a(ns0, ns1, cnt_n):
    nb = NP // NPB

    def body(ns0_ref, ns1_ref, cnt_ref, nm_ref, st_ref, acc):
        i = pl.program_id(0)
        c = cnt_ref[0, :, :1] + cnt_ref[1, :, :1]
        r = 1.0 / jnp.maximum(c, 1.0)
        nm = jnp.concatenate([(ns0_ref[0] + ns0_ref[1]) * r,
                              (ns1_ref[0] + ns1_ref[1]) * r], axis=1)
        nm_ref[...] = nm
        rows = i * NPB + lax.broadcasted_iota(jnp.int32, (NPB, 1), 0)
        nmm = jnp.where(rows < N, nm, 0.0)
        s1 = jnp.sum(nmm, axis=0)
        s2 = jnp.sum(nmm * nmm, axis=0)
        upd = jnp.concatenate(
            [s1[None, :], s2[None, :], jnp.zeros((6, 64), f32)], axis=0)

        @pl.when(i == 0)
        def _():
            acc[...] = jnp.zeros_like(acc)

        acc[...] += upd

        @pl.when(i == nb - 1)
        def _():
            st_ref[...] = acc[...]

    return pl.pallas_call(
        body,
        grid=(nb,),
        in_specs=[
            pl.BlockSpec((2, NPB, 32), lambda i: (0, i, 0)),
            pl.BlockSpec((2, NPB, 32), lambda i: (0, i, 0)),
            pl.BlockSpec((2, NPB, 32), lambda i: (0, i, 0)),
        ],
        out_specs=[
            pl.BlockSpec((NPB, 64), lambda i: (i, 0)),
            pl.BlockSpec((8, 64), lambda i: (0, 0)),
        ],
        out_shape=[
            jax.ShapeDtypeStruct((NP, 64), f32),
            jax.ShapeDtypeStruct((8, 64), f32),
        ],
        scratch_shapes=[pltpu.VMEM((8, 64), f32)],
    )(ns0, ns1, cnt_n)


def _t3b(nm, st, go, bo, h0, h1):
    def body(nm_ref, st_ref, g_ref, b_ref, h0_ref, h1_ref, o0_ref, o1_ref):
        mean = st_ref[0, :] * (1.0 / N)
        var = st_ref[1, :] * (1.0 / N) - mean * mean
        scale = g_ref[0, :] * lax.rsqrt(var + EPS)
        y = (nm_ref[...] - mean[None, :]) * scale[None, :] + b_ref[...]
        h = jnp.concatenate([h0_ref[...], h1_ref[...]], axis=1)
        hn = jax.nn.relu(jax.nn.softplus(y + h))
        o0_ref[...] = hn[:, :32]
        o1_ref[...] = hn[:, 32:]

    return pl.pallas_call(
        body,
        grid=(NP // NPB,),
        in_specs=[
            pl.BlockSpec((NPB, 64), lambda i: (i, 0)),
            pl.BlockSpec((8, 64), lambda i: (0, 0)),
            pl.BlockSpec((1, 64), lambda i: (0, 0)),
            pl.BlockSpec((1, 64), lambda i: (0, 0)),
            pl.BlockSpec((NPB, 32), lambda i: (i, 0)),
            pl.BlockSpec((NPB, 32), lambda i: (i, 0)),
        ],
        out_specs=[pl.BlockSpec((NPB, 32), lambda i: (i, 0))] * 2,
        out_shape=[jax.ShapeDtypeStruct((NP, 32), f32)] * 2,
    )(nm, st, go, bo, h0, h1)


def _t4_head(gs0, gs1, cnt_g, W2, b2, Wo, bo):
    def body(gs0_ref, gs1_ref, cnt_ref, w2_ref, b2_ref, wo_ref, bo_ref, o_ref):
        c = cnt_ref[0, :, :1] + cnt_ref[1, :, :1]
        r = 1.0 / jnp.maximum(c, 1.0)
        g = jnp.concatenate([(gs0_ref[0] + gs0_ref[1]) * r,
                             (gs1_ref[0] + gs1_ref[1]) * r], axis=1)
        t = jax.nn.softplus(
            jnp.dot(g, w2_ref[...], preferred_element_type=f32) + b2_ref[...])
        o_ref[...] = jnp.dot(t, wo_ref[...],
                             preferred_element_type=f32) + bo_ref[...]

    return pl.pallas_call(
        body,
        in_specs=[
            pl.BlockSpec((2, GACC, 32), lambda: (0, 0, 0)),
            pl.BlockSpec((2, GACC, 32), lambda: (0, 0, 0)),
            pl.BlockSpec((2, GACC, 32), lambda: (0, 0, 0)),
            pl.BlockSpec((64, 128), lambda: (0, 0)),
            pl.BlockSpec((1, 128), lambda: (0, 0)),
            pl.BlockSpec((128, 128), lambda: (0, 0)),
            pl.BlockSpec((1, 128), lambda: (0, 0)),
        ],
        out_specs=pl.BlockSpec((GACC, 128), lambda: (0, 0)),
        out_shape=jax.ShapeDtypeStruct((GACC, 128), f32),
    )(gs0, gs1, cnt_g, W2, b2, Wo, bo)


def kernel(x, hyperedge_index, hedge_attr, batch, params):
    nidx = hyperedge_index[0]
    hidx = hyperedge_index[1]
    nidx_g = jnp.pad(nidx, (0, EP - E)).reshape(EP // 128, 128)
    nidx_s = jnp.pad(nidx, (0, EP - E),
                     constant_values=N).reshape(EP // 128, 128)
    hidx_g = jnp.pad(hidx, (0, EP - E)).reshape(EP // 128, 128)
    hidx_s = jnp.pad(hidx, (0, EP - E),
                     constant_values=N).reshape(EP // 128, 128)
    batch_s = jnp.pad(batch, (0, NP - N),
                      constant_values=G).reshape(NP // 32, 32)

    x_p = jnp.pad(x, ((0, NP - N), (0, 0)))
    ha_p = jnp.pad(hedge_attr, ((0, NP - N), (0, 0)))
    zeros_np = jnp.zeros((NP, 32), f32)
    ones_e = jnp.ones((128, 32), f32)
    ones_g = jnp.ones((32, 32), f32)

    p = params
    WeT = p['embed']['W'].T
    be = p['embed']['b'][None, :]

    cnt_h = _sc_count(EP, 128, 4, 20, NP)(hidx_s, ones_e, zeros_np)
    cnt_n = _sc_count(EP, 128, 4, 20, NP)(nidx_s, ones_e, zeros_np)
    cnt_g = _sc_count(NP, 32, 7, 49, GACC)(batch_s, ones_g, zeros_np)

    h0, h1 = _t0_embed(x_p, WeT, be)

    for lp in p['layers']:
        W1 = jnp.concatenate([lp['lin_f1']['W'].T, lp['lin_c1']['W'].T], axis=1)
        b1 = jnp.concatenate([lp['lin_f1']['b'], lp['lin_c1']['b']])[None, :]
        g1 = jnp.concatenate([lp['bn_f']['g'], lp['bn_c']['g']])[None, :]
        be1 = jnp.concatenate([lp['bn_f']['b'], lp['bn_c']['b']])[None, :]
        WA = jnp.concatenate([lp['lin_f2']['W'][:, :64].T,
                              lp['lin_c2']['W'][:, :64].T], axis=1)
        bA = jnp.concatenate([lp['lin_f2']['b'], lp['lin_c2']['b']])[None, :]
        WB = jnp.concatenate([lp['lin_f2']['W'][:, 64:].T,
                              lp['lin_c2']['W'][:, 64:].T], axis=1)

        hs0 = _sc_gather_segsum(EP, 128, 4, 20, NP)(h0, nidx_g, hidx_s,
                                                    zeros_np)
        hs1 = _sc_gather_segsum(EP, 128, 4, 20, NP)(h1, nidx_g, hidx_s,
                                                    zeros_np)
        z, st = _t1a(hs0, hs1, cnt_h, ha_p, W1, b1)
        Afc, Bfc = _t1b(z, st, g1, be1, h0, h1, WA, bA, WB)
        EH = EP // 2
        HR = EH // 128
        An0, Bh0 = _sc_gather2(EH, 128, 2, 20)(Afc, Bfc,
                                               nidx_g[:HR], hidx_g[:HR])
        An1, Bh1 = _sc_gather2(EH, 128, 2, 20)(Afc, Bfc,
                                               nidx_g[HR:], hidx_g[HR:])
        m00, m10 = _t2_gate(An0, Bh0)
        m01, m11 = _t2_gate(An1, Bh1)
        ns00 = _sc_linear_segsum(EH, 128, 4, 20, NP)(m00, nidx_s[:HR],
                                                     zeros_np)
        ns01 = _sc_linear_segsum(EH, 128, 4, 20, NP)(m01, nidx_s[HR:],
                                                     zeros_np)
        ns10 = _sc_linear_segsum(EH, 128, 4, 20, NP)(m10, nidx_s[:HR],
                                                     zeros_np)
        ns11 = _sc_linear_segsum(EH, 128, 4, 20, NP)(m11, nidx_s[HR:],
                                                     zeros_np)
        nm, nst = _t3a(_t_sum2(ns00, ns01), _t_sum2(ns10, ns11), cnt_n)
        h0, h1 = _t3b(nm, nst, lp['bn_o']['g'][None, :],
                      lp['bn_o']['b'][None, :], h0, h1)

    gs0 = _sc_linear_segsum(NP, 32, 7, 49, GACC)(h0, batch_s, zeros_np)
    gs1 = _sc_linear_segsum(NP, 32, 7, 49, GACC)(h1, batch_s, zeros_np)

    W2 = p['l2']['W'].T
    b2 = p['l2']['b'][None, :]
    Wo = jnp.zeros((128, 128), f32).at[:, :1].set(p['out']['W'].T)
    bo = jnp.broadcast_to(p['out']['b'][None, :], (1, 128))

    out = _t4_head(gs0, gs1, cnt_g, W2, b2, Wo, bo)
    return out[:G, :1]


# speedup vs baseline: 1.2886x; 1.0664x over previous
import functools

import jax
import jax.numpy as jnp
from jax import lax
from jax.experimental import pallas as pl
from jax.experimental.pallas import tpu as pltpu
from jax.experimental.pallas import tpu_sc as plsc

N = 50000
NP = 50176
E = 800000
EP = 819200
G = 256
GACC = 272
EPS = 1e-5
NPB = 6272
EPB = 3200
f32 = jnp.float32

_mesh = lambda: plsc.VectorSubcoreMesh(core_axis_name="c", subcore_axis_name="s")
_SC_PARAMS = pltpu.CompilerParams(use_tc_tiling_on_sc=False)


@functools.lru_cache(maxsize=None)
def _sc_count(ep, k, ib, ibig, acc_rows):
    n_chunks = ep // (32 * k)
    stripe = acc_rows // 16

    @functools.partial(
        pl.kernel, mesh=_mesh(), compiler_params=_SC_PARAMS,
        out_type=jax.ShapeDtypeStruct((2, acc_rows, 32), f32),
        scratch_types=[
            pltpu.VMEM((ibig, k), jnp.int32),
            pltpu.VMEM((k, 32), f32),
            pltpu.VMEM_SHARED((acc_rows, 32), f32),
            pltpu.SemaphoreType.DMA,
        ],
    )
    def body(sidx2, ones, zeros, out, si2, ones_v, acc, sem):
        c = lax.axis_index("c")
        s = lax.axis_index("s")
        pltpu.sync_copy(zeros.at[pl.ds(s * stripe, stripe)],
                        acc.at[pl.ds(s * stripe, stripe)])
        pltpu.sync_copy(ones.at[pl.ds(0, k)], ones_v)
        plsc.subcore_barrier()
        chunk0 = (c * 16 + s) * n_chunks

        @pl.loop(0, n_chunks // ibig)
        def _(ob):
            blk0 = chunk0 + ob * ibig
            pltpu.sync_copy(sidx2.at[pl.ds(blk0, ibig)], si2)

            @pl.loop(0, ibig // ib)
            def _(g):
                cps = [pltpu.async_copy(ones_v, acc.at[si2.at[g * ib + b]],
                                        sem, add=True)
                       for b in range(ib)]
                for cp in cps:
                    cp.wait()

        plsc.subcore_barrier()
        pltpu.sync_copy(acc.at[pl.ds(s * stripe, stripe)],
                        out.at[c, pl.ds(s * stripe, stripe)])

    return body


@functools.lru_cache(maxsize=None)
def _sc_gather_segsum(ep, k, ib, ibig, acc_rows):
    n_chunks = ep // (32 * k)
    stripe = acc_rows // 16

    @functools.partial(
        pl.kernel, mesh=_mesh(), compiler_params=_SC_PARAMS,
        out_type=jax.ShapeDtypeStruct((2, acc_rows, 32), f32),
        scratch_types=[
            pltpu.VMEM((ibig, k), jnp.int32),
            pltpu.VMEM((ibig, k), jnp.int32),
            pltpu.VMEM((ib, k, 32), f32),
            pltpu.VMEM_SHARED((acc_rows, 32), f32),
            pltpu.SemaphoreType.DMA,
            pltpu.SemaphoreType.DMA,
            pltpu.SemaphoreType.DMA,
        ],
    )
    def body(table, gidx2, sidx2, zeros, out, gi2, si2, rows, acc,
             semi, semg, sems):
        c = lax.axis_index("c")
        s = lax.axis_index("s")
        pltpu.sync_copy(zeros.at[pl.ds(s * stripe, stripe)],
                        acc.at[pl.ds(s * stripe, stripe)])
        plsc.subcore_barrier()
        chunk0 = (c * 16 + s) * n_chunks
        n_inner = ibig // ib

        @pl.loop(0, n_chunks // ibig)
        def _(ob):
            @pl.when(ob > 0)
            def _():
                for b in range(ib):
                    pltpu.make_async_copy(
                        zeros.at[pl.ds(0, k)], rows.at[b], sems).wait()

            blk0 = chunk0 + ob * ibig
            cg = pltpu.async_copy(gidx2.at[pl.ds(blk0, ibig)], gi2, semi)
            cs = pltpu.async_copy(sidx2.at[pl.ds(blk0, ibig)], si2, semi)
            cg.wait()
            cs.wait()

            @pl.loop(0, n_inner)
            def _(g):
                @pl.when(g > 0)
                def _():
                    for b in range(ib):
                        pltpu.make_async_copy(
                            zeros.at[pl.ds(0, k)], rows.at[b], sems).wait()

                gs = [pltpu.async_copy(table.at[gi2.at[g * ib + b]],
                                       rows.at[b], semg)
                      for b in range(ib)]
                for b in range(ib):
                    gs[b].wait()
                    pltpu.async_copy(rows.at[b], acc.at[si2.at[g * ib + b]],
                                     sems, add=True)

        for b in range(ib):
            pltpu.make_async_copy(
                zeros.at[pl.ds(0, k)], rows.at[b], sems).wait()
        plsc.subcore_barrier()
        pltpu.sync_copy(acc.at[pl.ds(s * stripe, stripe)],
                        out.at[c, pl.ds(s * stripe, stripe)])

    return body


@functools.lru_cache(maxsize=None)
def _sc_linear_segsum(ep, k, ib, ibig, acc_rows):
    n_chunks = ep // (32 * k)
    stripe = acc_rows // 16

    @functools.partial(
        pl.kernel, mesh=_mesh(), compiler_params=_SC_PARAMS,
        out_type=jax.ShapeDtypeStruct((2, acc_rows, 32), f32),
        scratch_types=[
            pltpu.VMEM((ibig, k), jnp.int32),
            pltpu.VMEM((ib * k, 32), f32),
            pltpu.VMEM_SHARED((acc_rows, 32), f32),
            pltpu.SemaphoreType.DMA,
        ],
    )
    def body(rows2d, sidx2, zeros, out, si2, rows_v, acc, sem):
        c = lax.axis_index("c")
        s = lax.axis_index("s")
        pltpu.sync_copy(zeros.at[pl.ds(s * stripe, stripe)],
                        acc.at[pl.ds(s * stripe, stripe)])
        plsc.subcore_barrier()
        chunk0 = (c * 16 + s) * n_chunks

        @pl.loop(0, n_chunks // ibig)
        def _(ob):
            @pl.when(ob > 0)
            def _():
                for b in range(ib):
                    pltpu.make_async_copy(
                        zeros.at[pl.ds(0, k)],
                        rows_v.at[pl.ds(b * k, k)], sem).wait()

            blk0 = chunk0 + ob * ibig
            pltpu.sync_copy(sidx2.at[pl.ds(blk0, ibig)], si2)

            @pl.loop(0, ibig // ib)
            def _(g):
                @pl.when(g > 0)
                def _():
                    for b in range(ib):
                        pltpu.make_async_copy(
                            zeros.at[pl.ds(0, k)],
                            rows_v.at[pl.ds(b * k, k)], sem).wait()

                blk = blk0 + g * ib
                pltpu.sync_copy(rows2d.at[pl.ds(blk * k, ib * k)], rows_v)
                for b in range(ib):
                    pltpu.async_copy(rows_v.at[pl.ds(b * k, k)],
                                     acc.at[si2.at[g * ib + b]],
                                     sem, add=True)

        for b in range(ib):
            pltpu.make_async_copy(
                zeros.at[pl.ds(0, k)], rows_v.at[pl.ds(b * k, k)], sem).wait()
        plsc.subcore_barrier()
        pltpu.sync_copy(acc.at[pl.ds(s * stripe, stripe)],
                        out.at[c, pl.ds(s * stripe, stripe)])

    return body


@functools.lru_cache(maxsize=None)
def _sc_gather2(ep, k, ib, ibig):
    n_chunks = ep // (32 * k)

    @functools.partial(
        pl.kernel, mesh=_mesh(), compiler_params=_SC_PARAMS,
        out_type=(jax.ShapeDtypeStruct((ep, 128), f32),
                  jax.ShapeDtypeStruct((ep, 128), f32)),
        scratch_types=[
            pltpu.VMEM((ibig, k), jnp.int32),
            pltpu.VMEM((ibig, k), jnp.int32),
            pltpu.VMEM((ib, k, 128), f32),
            pltpu.VMEM((ib, k, 128), f32),
            pltpu.SemaphoreType.DMA,
            pltpu.SemaphoreType.DMA,
            pltpu.SemaphoreType.DMA,
        ],
    )
    def body(tA, tB, idxA2, idxB2, outA, outB, ia2, ib2, bufA, bufB,
             semi, semg, semw):
        c = lax.axis_index("c")
        s = lax.axis_index("s")
        chunk0 = (c * 16 + s) * n_chunks

        def drain_writes():
            for b in range(ib):
                pltpu.make_async_copy(
                    tA.at[pl.ds(0, k)], bufA.at[b], semw).wait()
                pltpu.make_async_copy(
                    tA.at[pl.ds(0, k)], bufB.at[b], semw).wait()

        @pl.loop(0, n_chunks // ibig)
        def _(ob):
            @pl.when(ob > 0)
            def _():
                drain_writes()

            blk0 = chunk0 + ob * ibig
            ca = pltpu.async_copy(idxA2.at[pl.ds(blk0, ibig)], ia2, semi)
            cb = pltpu.async_copy(idxB2.at[pl.ds(blk0, ibig)], ib2, semi)
            ca.wait()
            cb.wait()

            @pl.loop(0, ibig // ib)
            def _(g):
                @pl.when(g > 0)
                def _():
                    drain_writes()

                blk = blk0 + g * ib
                gs = []
                for b in range(ib):
                    gs.append(pltpu.async_copy(
                        tA.at[ia2.at[g * ib + b]], bufA.at[b], semg))
                    gs.append(pltpu.async_copy(
                        tB.at[ib2.at[g * ib + b]], bufB.at[b], semg))
                for b in range(ib):
                    gs[2 * b].wait()
                    pltpu.async_copy(
                        bufA.at[b], outA.at[pl.ds((blk + b) * k, k)], semw)
                    gs[2 * b + 1].wait()
                    pltpu.async_copy(
                        bufB.at[b], outB.at[pl.ds((blk + b) * k, k)], semw)

        drain_writes()

    return body


@functools.lru_cache(maxsize=None)
def _sc_count_dual(ep, k, ib, ibig, acc_rows):
    n_chunks = ep // (16 * k)
    stripe = acc_rows // 16

    @functools.partial(
        pl.kernel, mesh=_mesh(), compiler_params=_SC_PARAMS,
        out_type=jax.ShapeDtypeStruct((2, acc_rows, 32), f32),
        scratch_types=[
            pltpu.VMEM((ibig, k), jnp.int32),
            pltpu.VMEM((k, 32), f32),
            pltpu.VMEM_SHARED((acc_rows, 32), f32),
            pltpu.SemaphoreType.DMA,
        ],
    )
    def body(sidx2a, sidx2b, ones, zeros, out, si2, ones_v, acc, sem):
        c = lax.axis_index("c")
        s = lax.axis_index("s")
        pltpu.sync_copy(zeros.at[pl.ds(s * stripe, stripe)],
                        acc.at[pl.ds(s * stripe, stripe)])
        pltpu.sync_copy(ones.at[pl.ds(0, k)], ones_v)
        plsc.subcore_barrier()
        chunk0 = s * n_chunks

        def run(sidx2):
            @pl.loop(0, n_chunks // ibig)
            def _(ob):
                blk0 = chunk0 + ob * ibig
                pltpu.sync_copy(sidx2.at[pl.ds(blk0, ibig)], si2)

                @pl.loop(0, ibig // ib)
                def _(g):
                    cps = [pltpu.async_copy(ones_v,
                                            acc.at[si2.at[g * ib + b]],
                                            sem, add=True)
                           for b in range(ib)]
                    for cp in cps:
                        cp.wait()

        @pl.when(c == 0)
        def _():
            run(sidx2a)

        @pl.when(c == 1)
        def _():
            run(sidx2b)

        plsc.subcore_barrier()
        pltpu.sync_copy(acc.at[pl.ds(s * stripe, stripe)],
                        out.at[c, pl.ds(s * stripe, stripe)])

    return body


@functools.lru_cache(maxsize=None)
def _sc_gather_segsum_dual(ep, k, ib, ibig, acc_rows):
    n_chunks = ep // (16 * k)
    stripe = acc_rows // 16

    @functools.partial(
        pl.kernel, mesh=_mesh(), compiler_params=_SC_PARAMS,
        out_type=jax.ShapeDtypeStruct((2, acc_rows, 32), f32),
        scratch_types=[
            pltpu.VMEM((ibig, k), jnp.int32),
            pltpu.VMEM((ibig, k), jnp.int32),
            pltpu.VMEM((ib, k, 32), f32),
            pltpu.VMEM_SHARED((acc_rows, 32), f32),
            pltpu.SemaphoreType.DMA,
            pltpu.SemaphoreType.DMA,
            pltpu.SemaphoreType.DMA,
        ],
    )
    def body(t0, t1, gidx2, sidx2, zeros, out, gi2, si2, rows, acc,
             semi, semg, sems):
        c = lax.axis_index("c")
        s = lax.axis_index("s")
        pltpu.sync_copy(zeros.at[pl.ds(s * stripe, stripe)],
                        acc.at[pl.ds(s * stripe, stripe)])
        plsc.subcore_barrier()
        chunk0 = s * n_chunks

        def run(table):
            @pl.loop(0, n_chunks // ibig)
            def _(ob):
                @pl.when(ob > 0)
                def _():
                    for b in range(ib):
                        pltpu.make_async_copy(
                            zeros.at[pl.ds(0, k)], rows.at[b], sems).wait()

                blk0 = chunk0 + ob * ibig
                cg = pltpu.async_copy(gidx2.at[pl.ds(blk0, ibig)], gi2, semi)
                cs = pltpu.async_copy(sidx2.at[pl.ds(blk0, ibig)], si2, semi)
                cg.wait()
                cs.wait()

                @pl.loop(0, ibig // ib)
                def _(g):
                    @pl.when(g > 0)
                    def _():
                        for b in range(ib):
                            pltpu.make_async_copy(
                                zeros.at[pl.ds(0, k)], rows.at[b],
                                sems).wait()

                    gs = [pltpu.async_copy(table.at[gi2.at[g * ib + b]],
                                           rows.at[b], semg)
                          for b in range(ib)]
                    for b in range(ib):
                        gs[b].wait()
                        pltpu.async_copy(rows.at[b],
                                         acc.at[si2.at[g * ib + b]],
                                         sems, add=True)

            for b in range(ib):
                pltpu.make_async_copy(
                    zeros.at[pl.ds(0, k)], rows.at[b], sems).wait()

        @pl.when(c == 0)
        def _():
            run(t0)

        @pl.when(c == 1)
        def _():
            run(t1)

        plsc.subcore_barrier()
        pltpu.sync_copy(acc.at[pl.ds(s * stripe, stripe)],
                        out.at[c, pl.ds(s * stripe, stripe)])

    return body


@functools.lru_cache(maxsize=None)
def _sc_linear_segsum_dual(ep, k, ib, ibig, acc_rows):
    n_chunks = ep // (16 * k)
    stripe = acc_rows // 16

    @functools.partial(
        pl.kernel, mesh=_mesh(), compiler_params=_SC_PARAMS,
        out_type=jax.ShapeDtypeStruct((2, acc_rows, 32), f32),
        scratch_types=[
            pltpu.VMEM((ibig, k), jnp.int32),
            pltpu.VMEM((ib * k, 32), f32),
            pltpu.VMEM_SHARED((acc_rows, 32), f32),
            pltpu.SemaphoreType.DMA,
        ],
    )
    def body(r0, r1, sidx2, zeros, out, si2, rows_v, acc, sem):
        c = lax.axis_index("c")
        s = lax.axis_index("s")
        pltpu.sync_copy(zeros.at[pl.ds(s * stripe, stripe)],
                        acc.at[pl.ds(s * stripe, stripe)])
        plsc.subcore_barrier()
        chunk0 = s * n_chunks

        def run(rows2d):
            @pl.loop(0, n_chunks // ibig)
            def _(ob):
                @pl.when(ob > 0)
                def _():
                    for b in range(ib):
                        pltpu.make_async_copy(
                            zeros.at[pl.ds(0, k)],
                            rows_v.at[pl.ds(b * k, k)], sem).wait()

                blk0 = chunk0 + ob * ibig
                pltpu.sync_copy(sidx2.at[pl.ds(blk0, ibig)], si2)

                @pl.loop(0, ibig // ib)
                def _(g):
                    @pl.when(g > 0)
                    def _():
                        for b in range(ib):
                            pltpu.make_async_copy(
                                zeros.at[pl.ds(0, k)],
                                rows_v.at[pl.ds(b * k, k)], sem).wait()

                    blk = blk0 + g * ib
                    pltpu.sync_copy(rows2d.at[pl.ds(blk * k, ib * k)], rows_v)
                    for b in range(ib):
                        pltpu.async_copy(rows_v.at[pl.ds(b * k, k)],
                                         acc.at[si2.at[g * ib + b]],
                                         sem, add=True)

            for b in range(ib):
                pltpu.make_async_copy(
                    zeros.at[pl.ds(0, k)],
                    rows_v.at[pl.ds(b * k, k)], sem).wait()

        @pl.when(c == 0)
        def _():
            run(r0)

        @pl.when(c == 1)
        def _():
            run(r1)

        plsc.subcore_barrier()
        pltpu.sync_copy(acc.at[pl.ds(s * stripe, stripe)],
                        out.at[c, pl.ds(s * stripe, stripe)])

    return body


def _t0_embed(x_p, WeT, be):
    def body(x_ref, w_ref, b_ref, h0_ref, h1_ref):
        h = jnp.dot(x_ref[...], w_ref[...], preferred_element_type=f32) + b_ref[...]
        h0_ref[...] = h[:, :32]
        h1_ref[...] = h[:, 32:]

    return pl.pallas_call(
        body,
        grid=(NP // NPB,),
        in_specs=[
            pl.BlockSpec((NPB, 92), lambda i: (i, 0)),
            pl.BlockSpec((92, 64), lambda i: (0, 0)),
            pl.BlockSpec((1, 64), lambda i: (0, 0)),
        ],
        out_specs=[pl.BlockSpec((NPB, 32), lambda i: (i, 0))] * 2,
        out_shape=[jax.ShapeDtypeStruct((NP, 32), f32)] * 2,
    )(x_p, WeT, be)


def _t1a(hs, cnt_hn, ha_p, W1, b1):
    nb = NP // NPB

    def body(hs_ref, cnt_ref, ha_ref, w_ref, b_ref,
             z_ref, st_ref, acc):
        i = pl.program_id(0)
        r = 1.0 / jnp.maximum(cnt_ref[0, :, :1], 1.0)
        hm0 = hs_ref[0] * r
        hm1 = hs_ref[1] * r
        msg = jnp.concatenate([hm0, hm1, ha_ref[...]], axis=1)
        z = jnp.dot(msg, w_ref[...], preferred_element_type=f32) + b_ref[...]
        z_ref[...] = z
        rows = i * NPB + lax.broadcasted_iota(jnp.int32, (NPB, 1), 0)
        zm = jnp.where(rows < N, z, 0.0)
        s1 = jnp.sum(zm, axis=0)
        s2 = jnp.sum(zm * zm, axis=0)
        upd = jnp.concatenate(
            [s1[None, :], s2[None, :], jnp.zeros((6, 70), f32)], axis=0)

        @pl.when(i == 0)
        def _():
            acc[...] = jnp.zeros_like(acc)

        acc[...] += upd

        @pl.when(i == nb - 1)
        def _():
            st_ref[...] = acc[...]

    return pl.pallas_call(
        body,
        grid=(nb,),
        in_specs=[
            pl.BlockSpec((2, NPB, 32), lambda i: (0, i, 0)),
            pl.BlockSpec((2, NPB, 32), lambda i: (0, i, 0)),
            pl.BlockSpec((NPB, 35), lambda i: (i, 0)),
            pl.BlockSpec((99, 70), lambda i: (0, 0)),
            pl.BlockSpec((1, 70), lambda i: (0, 0)),
        ],
        out_specs=[
            pl.BlockSpec((NPB, 70), lambda i: (i, 0)),
            pl.BlockSpec((8, 70), lambda i: (0, 0)),
        ],
        out_shape=[
            jax.ShapeDtypeStruct((NP, 70), f32),
            jax.ShapeDtypeStruct((8, 70), f32),
        ],
        scratch_shapes=[pltpu.VMEM((8, 70), f32)],
    )(hs, cnt_hn, ha_p, W1, b1)


def _t1b(z, st, g1, be1, h0, h1, WA, bA, WB):
    def body(z_ref, st_ref, g_ref, be_ref, h0_ref, h1_ref,
             wa_ref, ba_ref, wb_ref, afc_ref, bfc_ref):
        mean = st_ref[0, :] * (1.0 / N)
        var = st_ref[1, :] * (1.0 / N) - mean * mean
        scale = g_ref[0, :] * lax.rsqrt(var + EPS)
        zn = (z_ref[...] - mean[None, :]) * scale[None, :] + be_ref[...]
        ha = jax.nn.sigmoid(zn[:, :35]) * jax.nn.softplus(zn[:, 35:70])
        bfc_ref[...] = jnp.dot(ha, wb_ref[...], preferred_element_type=f32)
        h = jnp.concatenate([h0_ref[...], h1_ref[...]], axis=1)
        afc_ref[...] = jnp.dot(h, wa_ref[...],
                               preferred_element_type=f32) + ba_ref[...]

    return pl.pallas_call(
        body,
        grid=(NP // NPB,),
        in_specs=[
            pl.BlockSpec((NPB, 70), lambda i: (i, 0)),
            pl.BlockSpec((8, 70), lambda i: (0, 0)),
            pl.BlockSpec((1, 70), lambda i: (0, 0)),
            pl.BlockSpec((1, 70), lambda i: (0, 0)),
            pl.BlockSpec((NPB, 32), lambda i: (i, 0)),
            pl.BlockSpec((NPB, 32), lambda i: (i, 0)),
            pl.BlockSpec((64, 128), lambda i: (0, 0)),
            pl.BlockSpec((1, 128), lambda i: (0, 0)),
            pl.BlockSpec((35, 128), lambda i: (0, 0)),
        ],
        out_specs=[
            pl.BlockSpec((NPB, 128), lambda i: (i, 0)),
            pl.BlockSpec((NPB, 128), lambda i: (i, 0)),
        ],
        out_shape=[jax.ShapeDtypeStruct((NP, 128), f32)] * 2,
    )(z, st, g1, be1, h0, h1, WA, bA, WB)


def _t2_gate(An, Bh):
    rows = An.shape[0]

    def body(a_ref, b_ref, m0_ref, m1_ref):
        e = a_ref[...] + b_ref[...]
        m = jax.nn.sigmoid(e[:, :64]) * jax.nn.softplus(e[:, 64:])
        m0_ref[...] = m[:, :32]
        m1_ref[...] = m[:, 32:]

    return pl.pallas_call(
        body,
        grid=(rows // EPB,),
        in_specs=[
            pl.BlockSpec((EPB, 128), lambda i: (i, 0)),
            pl.BlockSpec((EPB, 128), lambda i: (i, 0)),
        ],
        out_specs=[pl.BlockSpec((EPB, 32), lambda i: (i, 0))] * 2,
        out_shape=[jax.ShapeDtypeStruct((rows, 32), f32)] * 2,
    )(An, Bh)


def _t_sum2(a, b):
    av = a.reshape(2, NP // 4, 128)
    bv = b.reshape(2, NP // 4, 128)
    rb = NP // 4 // 8

    def body(a_ref, b_ref, o_ref):
        o_ref[...] = a_ref[...] + b_ref[...]

    out = pl.pallas_call(
        body,
        grid=(8,),
        in_specs=[pl.BlockSpec((2, rb, 128), lambda i: (0, i, 0))] * 2,
        out_specs=pl.BlockSpec((2, rb, 128), lambda i: (0, i, 0)),
        out_shape=jax.ShapeDtypeStruct((2, NP // 4, 128), f32),
    )(av, bv)
    return out.reshape(2, NP, 32)


def _t3a(ns, cnt_hn):
    nb = NP // NPB

    def body(ns_ref, cnt_ref, nm_ref, st_ref, acc):
        i = pl.program_id(0)
        r = 1.0 / jnp.maximum(cnt_ref[1, :, :1], 1.0)
        nm = jnp.concatenate([ns_ref[0] * r, ns_ref[1] * r], axis=1)
        nm_ref[...] = nm
        rows = i * NPB + lax.broadcasted_iota(jnp.int32, (NPB, 1), 0)
        nmm = jnp.where(rows < N, nm, 0.0)
        s1 = jnp.sum(nmm, axis=0)
        s2 = jnp.sum(nmm * nmm, axis=0)
        upd = jnp.concatenate(
            [s1[None, :], s2[None, :], jnp.zeros((6, 64), f32)], axis=0)

        @pl.when(i == 0)
        def _():
            acc[...] = jnp.zeros_like(acc)

        acc[...] += upd

        @pl.when(i == nb - 1)
        def _():
            st_ref[...] = acc[...]

    return pl.pallas_call(
        body,
        grid=(nb,),
        in_specs=[
            pl.BlockSpec((2, NPB, 32), lambda i: (0, i, 0)),
            pl.BlockSpec((2, NPB, 32), lambda i: (0, i, 0)),
        ],
        out_specs=[
            pl.BlockSpec((NPB, 64), lambda i: (i, 0)),
            pl.BlockSpec((8, 64), lambda i: (0, 0)),
        ],
        out_shape=[
            jax.ShapeDtypeStruct((NP, 64), f32),
            jax.ShapeDtypeStruct((8, 64), f32),
        ],
        scratch_shapes=[pltpu.VMEM((8, 64), f32)],
    )(ns, cnt_hn)


def _t3b(nm, st, go, bo, h0, h1):
    def body(nm_ref, st_ref, g_ref, b_ref, h0_ref, h1_ref, o0_ref, o1_ref):
        mean = st_ref[0, :] * (1.0 / N)
        var = st_ref[1, :] * (1.0 / N) - mean * mean
        scale = g_ref[0, :] * lax.rsqrt(var + EPS)
        y = (nm_ref[...] - mean[None, :]) * scale[None, :] + b_ref[...]
        h = jnp.concatenate([h0_ref[...], h1_ref[...]], axis=1)
        hn = jax.nn.relu(jax.nn.softplus(y + h))
        o0_ref[...] = hn[:, :32]
        o1_ref[...] = hn[:, 32:]

    return pl.pallas_call(
        body,
        grid=(NP // NPB,),
        in_specs=[
            pl.BlockSpec((NPB, 64), lambda i: (i, 0)),
            pl.BlockSpec((8, 64), lambda i: (0, 0)),
            pl.BlockSpec((1, 64), lambda i: (0, 0)),
            pl.BlockSpec((1, 64), lambda i: (0, 0)),
            pl.BlockSpec((NPB, 32), lambda i: (i, 0)),
            pl.BlockSpec((NPB, 32), lambda i: (i, 0)),
        ],
        out_specs=[pl.BlockSpec((NPB, 32), lambda i: (i, 0))] * 2,
        out_shape=[jax.ShapeDtypeStruct((NP, 32), f32)] * 2,
    )(nm, st, go, bo, h0, h1)


def _t4_head(gs, cnt_g, W2, b2, Wo, bo):
    def body(gs_ref, cnt_ref, w2_ref, b2_ref, wo_ref, bo_ref, o_ref):
        c = cnt_ref[0, :, :1] + cnt_ref[1, :, :1]
        r = 1.0 / jnp.maximum(c, 1.0)
        g = jnp.concatenate([gs_ref[0] * r, gs_ref[1] * r], axis=1)
        t = jax.nn.softplus(
            jnp.dot(g, w2_ref[...], preferred_element_type=f32) + b2_ref[...])
        o_ref[...] = jnp.dot(t, wo_ref[...],
                             preferred_element_type=f32) + bo_ref[...]

    return pl.pallas_call(
        body,
        in_specs=[
            pl.BlockSpec((2, GACC, 32), lambda: (0, 0, 0)),
            pl.BlockSpec((2, GACC, 32), lambda: (0, 0, 0)),
            pl.BlockSpec((64, 128), lambda: (0, 0)),
            pl.BlockSpec((1, 128), lambda: (0, 0)),
            pl.BlockSpec((128, 128), lambda: (0, 0)),
            pl.BlockSpec((1, 128), lambda: (0, 0)),
        ],
        out_specs=pl.BlockSpec((GACC, 128), lambda: (0, 0)),
        out_shape=jax.ShapeDtypeStruct((GACC, 128), f32),
    )(gs, cnt_g, W2, b2, Wo, bo)


def kernel(x, hyperedge_index, hedge_attr, batch, params):
    nidx = hyperedge_index[0]
    hidx = hyperedge_index[1]
    nidx_g = jnp.pad(nidx, (0, EP - E)).reshape(EP // 128, 128)
    nidx_s = jnp.pad(nidx, (0, EP - E),
                     constant_values=N).reshape(EP // 128, 128)
    hidx_g = jnp.pad(hidx, (0, EP - E)).reshape(EP // 128, 128)
    hidx_s = jnp.pad(hidx, (0, EP - E),
                     constant_values=N).reshape(EP // 128, 128)
    batch_s = jnp.pad(batch, (0, NP - N),
                      constant_values=G).reshape(NP // 32, 32)

    x_p = jnp.pad(x, ((0, NP - N), (0, 0)))
    ha_p = jnp.pad(hedge_attr, ((0, NP - N), (0, 0)))
    zeros_np = jnp.zeros((NP, 32), f32)
    ones_e = jnp.ones((128, 32), f32)
    ones_g = jnp.ones((32, 32), f32)

    p = params
    WeT = p['embed']['W'].T
    be = p['embed']['b'][None, :]

    cnt_hn = _sc_count_dual(EP, 128, 4, 20, NP)(hidx_s, nidx_s,
                                                ones_e, zeros_np)
    cnt_g = _sc_count(NP, 32, 7, 49, GACC)(batch_s, ones_g, zeros_np)

    h0, h1 = _t0_embed(x_p, WeT, be)

    for lp in p['layers']:
        W1 = jnp.concatenate([lp['lin_f1']['W'].T, lp['lin_c1']['W'].T], axis=1)
        b1 = jnp.concatenate([lp['lin_f1']['b'], lp['lin_c1']['b']])[None, :]
        g1 = jnp.concatenate([lp['bn_f']['g'], lp['bn_c']['g']])[None, :]
        be1 = jnp.concatenate([lp['bn_f']['b'], lp['bn_c']['b']])[None, :]
        WA = jnp.concatenate([lp['lin_f2']['W'][:, :64].T,
                              lp['lin_c2']['W'][:, :64].T], axis=1)
        bA = jnp.concatenate([lp['lin_f2']['b'], lp['lin_c2']['b']])[None, :]
        WB = jnp.concatenate([lp['lin_f2']['W'][:, 64:].T,
                              lp['lin_c2']['W'][:, 64:].T], axis=1)

        hs = _sc_gather_segsum_dual(EP, 128, 4, 20, NP)(h0, h1, nidx_g,
                                                        hidx_s, zeros_np)
        z, st = _t1a(hs, cnt_hn, ha_p, W1, b1)
        Afc, Bfc = _t1b(z, st, g1, be1, h0, h1, WA, bA, WB)
        EH = EP // 2
        HR = EH // 128
        An0, Bh0 = _sc_gather2(EH, 128, 2, 20)(Afc, Bfc,
                                               nidx_g[:HR], hidx_g[:HR])
        An1, Bh1 = _sc_gather2(EH, 128, 2, 20)(Afc, Bfc,
                                               nidx_g[HR:], hidx_g[HR:])
        m00, m10 = _t2_gate(An0, Bh0)
        m01, m11 = _t2_gate(An1, Bh1)
        nsa = _sc_linear_segsum_dual(EH, 128, 4, 20, NP)(m00, m10,
                                                         nidx_s[:HR],
                                                         zeros_np)
        nsb = _sc_linear_segsum_dual(EH, 128, 4, 20, NP)(m01, m11,
                                                         nidx_s[HR:],
                                                         zeros_np)
        nm, nst = _t3a(_t_sum2(nsa, nsb), cnt_hn)
        h0, h1 = _t3b(nm, nst, lp['bn_o']['g'][None, :],
                      lp['bn_o']['b'][None, :], h0, h1)

    gs = _sc_linear_segsum_dual(NP, 32, 7, 14, GACC)(h0, h1, batch_s,
                                                     zeros_np)

    W2 = p['l2']['W'].T
    b2 = p['l2']['b'][None, :]
    Wo = jnp.zeros((128, 128), f32).at[:, :1].set(p['out']['W'].T)
    bo = jnp.broadcast_to(p['out']['b'][None, :], (1, 128))

    out = _t4_head(gs, cnt_g, W2, b2, Wo, bo)
    return out[:G, :1]


# speedup vs baseline: 1.3307x; 1.0326x over previous
import functools

import jax
import jax.numpy as jnp
from jax import lax
from jax.experimental import pallas as pl
from jax.experimental.pallas import tpu as pltpu
from jax.experimental.pallas import tpu_sc as plsc

N = 50000
NP = 50176
E = 800000
EP = 819200
G = 256
GACC = 272
EPS = 1e-5
NPB = 6272
EPB = 3200
f32 = jnp.float32

_mesh = lambda: plsc.VectorSubcoreMesh(core_axis_name="c", subcore_axis_name="s")
_SC_PARAMS = pltpu.CompilerParams(use_tc_tiling_on_sc=False)


@functools.lru_cache(maxsize=None)
def _sc_count(ep, k, ib, ibig, acc_rows):
    n_chunks = ep // (32 * k)
    stripe = acc_rows // 16

    @functools.partial(
        pl.kernel, mesh=_mesh(), compiler_params=_SC_PARAMS,
        out_type=jax.ShapeDtypeStruct((2, acc_rows, 32), f32),
        scratch_types=[
            pltpu.VMEM((ibig, k), jnp.int32),
            pltpu.VMEM((k, 32), f32),
            pltpu.VMEM_SHARED((acc_rows, 32), f32),
            pltpu.SemaphoreType.DMA,
        ],
    )
    def body(sidx2, ones, zeros, out, si2, ones_v, acc, sem):
        c = lax.axis_index("c")
        s = lax.axis_index("s")
        pltpu.sync_copy(zeros.at[pl.ds(s * stripe, stripe)],
                        acc.at[pl.ds(s * stripe, stripe)])
        pltpu.sync_copy(ones.at[pl.ds(0, k)], ones_v)
        plsc.subcore_barrier()
        chunk0 = (c * 16 + s) * n_chunks

        @pl.loop(0, n_chunks // ibig)
        def _(ob):
            blk0 = chunk0 + ob * ibig
            pltpu.sync_copy(sidx2.at[pl.ds(blk0, ibig)], si2)

            @pl.loop(0, ibig // ib)
            def _(g):
                cps = [pltpu.async_copy(ones_v, acc.at[si2.at[g * ib + b]],
                                        sem, add=True)
                       for b in range(ib)]
                for cp in cps:
                    cp.wait()

        plsc.subcore_barrier()
        pltpu.sync_copy(acc.at[pl.ds(s * stripe, stripe)],
                        out.at[c, pl.ds(s * stripe, stripe)])

    return body


@functools.lru_cache(maxsize=None)
def _sc_gather_segsum(ep, k, ib, ibig, acc_rows):
    n_chunks = ep // (32 * k)
    stripe = acc_rows // 16

    @functools.partial(
        pl.kernel, mesh=_mesh(), compiler_params=_SC_PARAMS,
        out_type=jax.ShapeDtypeStruct((2, acc_rows, 32), f32),
        scratch_types=[
            pltpu.VMEM((ibig, k), jnp.int32),
            pltpu.VMEM((ibig, k), jnp.int32),
            pltpu.VMEM((ib, k, 32), f32),
            pltpu.VMEM_SHARED((acc_rows, 32), f32),
            pltpu.SemaphoreType.DMA,
            pltpu.SemaphoreType.DMA,
            pltpu.SemaphoreType.DMA,
        ],
    )
    def body(table, gidx2, sidx2, zeros, out, gi2, si2, rows, acc,
             semi, semg, sems):
        c = lax.axis_index("c")
        s = lax.axis_index("s")
        pltpu.sync_copy(zeros.at[pl.ds(s * stripe, stripe)],
                        acc.at[pl.ds(s * stripe, stripe)])
        plsc.subcore_barrier()
        chunk0 = (c * 16 + s) * n_chunks
        n_inner = ibig // ib

        @pl.loop(0, n_chunks // ibig)
        def _(ob):
            @pl.when(ob > 0)
            def _():
                for b in range(ib):
                    pltpu.make_async_copy(
                        zeros.at[pl.ds(0, k)], rows.at[b], sems).wait()

            blk0 = chunk0 + ob * ibig
            cg = pltpu.async_copy(gidx2.at[pl.ds(blk0, ibig)], gi2, semi)
            cs = pltpu.async_copy(sidx2.at[pl.ds(blk0, ibig)], si2, semi)
            cg.wait()
            cs.wait()

            @pl.loop(0, n_inner)
            def _(g):
                @pl.when(g > 0)
                def _():
                    for b in range(ib):
                        pltpu.make_async_copy(
                            zeros.at[pl.ds(0, k)], rows.at[b], sems).wait()

                gs = [pltpu.async_copy(table.at[gi2.at[g * ib + b]],
                                       rows.at[b], semg)
                      for b in range(ib)]
                for b in range(ib):
                    gs[b].wait()
                    pltpu.async_copy(rows.at[b], acc.at[si2.at[g * ib + b]],
                                     sems, add=True)

        for b in range(ib):
            pltpu.make_async_copy(
                zeros.at[pl.ds(0, k)], rows.at[b], sems).wait()
        plsc.subcore_barrier()
        pltpu.sync_copy(acc.at[pl.ds(s * stripe, stripe)],
                        out.at[c, pl.ds(s * stripe, stripe)])

    return body


@functools.lru_cache(maxsize=None)
def _sc_linear_segsum(ep, k, ib, ibig, acc_rows):
    n_chunks = ep // (32 * k)
    stripe = acc_rows // 16

    @functools.partial(
        pl.kernel, mesh=_mesh(), compiler_params=_SC_PARAMS,
        out_type=jax.ShapeDtypeStruct((2, acc_rows, 32), f32),
        scratch_types=[
            pltpu.VMEM((ibig, k), jnp.int32),
            pltpu.VMEM((ib * k, 32), f32),
            pltpu.VMEM_SHARED((acc_rows, 32), f32),
            pltpu.SemaphoreType.DMA,
        ],
    )
    def body(rows2d, sidx2, zeros, out, si2, rows_v, acc, sem):
        c = lax.axis_index("c")
        s = lax.axis_index("s")
        pltpu.sync_copy(zeros.at[pl.ds(s * stripe, stripe)],
                        acc.at[pl.ds(s * stripe, stripe)])
        plsc.subcore_barrier()
        chunk0 = (c * 16 + s) * n_chunks

        @pl.loop(0, n_chunks // ibig)
        def _(ob):
            @pl.when(ob > 0)
            def _():
                for b in range(ib):
                    pltpu.make_async_copy(
                        zeros.at[pl.ds(0, k)],
                        rows_v.at[pl.ds(b * k, k)], sem).wait()

            blk0 = chunk0 + ob * ibig
            pltpu.sync_copy(sidx2.at[pl.ds(blk0, ibig)], si2)

            @pl.loop(0, ibig // ib)
            def _(g):
                @pl.when(g > 0)
                def _():
                    for b in range(ib):
                        pltpu.make_async_copy(
                            zeros.at[pl.ds(0, k)],
                            rows_v.at[pl.ds(b * k, k)], sem).wait()

                blk = blk0 + g * ib
                pltpu.sync_copy(rows2d.at[pl.ds(blk * k, ib * k)], rows_v)
                for b in range(ib):
                    pltpu.async_copy(rows_v.at[pl.ds(b * k, k)],
                                     acc.at[si2.at[g * ib + b]],
                                     sem, add=True)

        for b in range(ib):
            pltpu.make_async_copy(
                zeros.at[pl.ds(0, k)], rows_v.at[pl.ds(b * k, k)], sem).wait()
        plsc.subcore_barrier()
        pltpu.sync_copy(acc.at[pl.ds(s * stripe, stripe)],
                        out.at[c, pl.ds(s * stripe, stripe)])

    return body


@functools.lru_cache(maxsize=None)
def _sc_gather2(ep, k, ib, ibig):
    n_chunks = ep // (32 * k)

    @functools.partial(
        pl.kernel, mesh=_mesh(), compiler_params=_SC_PARAMS,
        out_type=(jax.ShapeDtypeStruct((ep, 128), f32),
                  jax.ShapeDtypeStruct((ep, 128), f32)),
        scratch_types=[
            pltpu.VMEM((ibig, k), jnp.int32),
            pltpu.VMEM((ibig, k), jnp.int32),
            pltpu.VMEM((ib, k, 128), f32),
            pltpu.VMEM((ib, k, 128), f32),
            pltpu.SemaphoreType.DMA,
            pltpu.SemaphoreType.DMA,
            pltpu.SemaphoreType.DMA,
        ],
    )
    def body(tA, tB, idxA2, idxB2, outA, outB, ia2, ib2, bufA, bufB,
             semi, semg, semw):
        c = lax.axis_index("c")
        s = lax.axis_index("s")
        chunk0 = (c * 16 + s) * n_chunks

        def drain_writes():
            for b in range(ib):
                pltpu.make_async_copy(
                    tA.at[pl.ds(0, k)], bufA.at[b], semw).wait()
                pltpu.make_async_copy(
                    tA.at[pl.ds(0, k)], bufB.at[b], semw).wait()

        @pl.loop(0, n_chunks // ibig)
        def _(ob):
            @pl.when(ob > 0)
            def _():
                drain_writes()

            blk0 = chunk0 + ob * ibig
            ca = pltpu.async_copy(idxA2.at[pl.ds(blk0, ibig)], ia2, semi)
            cb = pltpu.async_copy(idxB2.at[pl.ds(blk0, ibig)], ib2, semi)
            ca.wait()
            cb.wait()

            @pl.loop(0, ibig // ib)
            def _(g):
                @pl.when(g > 0)
                def _():
                    drain_writes()

                blk = blk0 + g * ib
                gs = []
                for b in range(ib):
                    gs.append(pltpu.async_copy(
                        tA.at[ia2.at[g * ib + b]], bufA.at[b], semg))
                    gs.append(pltpu.async_copy(
                        tB.at[ib2.at[g * ib + b]], bufB.at[b], semg))
                for b in range(ib):
                    gs[2 * b].wait()
                    pltpu.async_copy(
                        bufA.at[b], outA.at[pl.ds((blk + b) * k, k)], semw)
                    gs[2 * b + 1].wait()
                    pltpu.async_copy(
                        bufB.at[b], outB.at[pl.ds((blk + b) * k, k)], semw)

        drain_writes()

    return body


@functools.lru_cache(maxsize=None)
def _sc_count_dual(ep, k, ib, ibig, acc_rows):
    n_chunks = ep // (16 * k)
    stripe = acc_rows // 16

    @functools.partial(
        pl.kernel, mesh=_mesh(), compiler_params=_SC_PARAMS,
        out_type=jax.ShapeDtypeStruct((2, acc_rows, 32), f32),
        scratch_types=[
            pltpu.VMEM((ibig, k), jnp.int32),
            pltpu.VMEM((k, 32), f32),
            pltpu.VMEM_SHARED((acc_rows, 32), f32),
            pltpu.SemaphoreType.DMA,
        ],
    )
    def body(sidx2a, sidx2b, ones, zeros, out, si2, ones_v, acc, sem):
        c = lax.axis_index("c")
        s = lax.axis_index("s")
        pltpu.sync_copy(zeros.at[pl.ds(s * stripe, stripe)],
                        acc.at[pl.ds(s * stripe, stripe)])
        pltpu.sync_copy(ones.at[pl.ds(0, k)], ones_v)
        plsc.subcore_barrier()
        chunk0 = s * n_chunks

        def run(sidx2):
            @pl.loop(0, n_chunks // ibig)
            def _(ob):
                blk0 = chunk0 + ob * ibig
                pltpu.sync_copy(sidx2.at[pl.ds(blk0, ibig)], si2)

                @pl.loop(0, ibig // ib)
                def _(g):
                    cps = [pltpu.async_copy(ones_v,
                                            acc.at[si2.at[g * ib + b]],
                                            sem, add=True)
                           for b in range(ib)]
                    for cp in cps:
                        cp.wait()

        @pl.when(c == 0)
        def _():
            run(sidx2a)

        @pl.when(c == 1)
        def _():
            run(sidx2b)

        plsc.subcore_barrier()
        pltpu.sync_copy(acc.at[pl.ds(s * stripe, stripe)],
                        out.at[c, pl.ds(s * stripe, stripe)])

    return body


@functools.lru_cache(maxsize=None)
def _sc_gather_segsum_dual(ep, k, ib, ibig, acc_rows):
    n_chunks = ep // (16 * k)
    stripe = acc_rows // 16

    @functools.partial(
        pl.kernel, mesh=_mesh(), compiler_params=_SC_PARAMS,
        out_type=jax.ShapeDtypeStruct((2, acc_rows, 32), f32),
        scratch_types=[
            pltpu.VMEM((ibig, k), jnp.int32),
            pltpu.VMEM((ibig, k), jnp.int32),
            pltpu.VMEM((ib, k, 32), f32),
            pltpu.VMEM_SHARED((acc_rows, 32), f32),
            pltpu.SemaphoreType.DMA,
            pltpu.SemaphoreType.DMA,
            pltpu.SemaphoreType.DMA,
        ],
    )
    def body(t0, t1, gidx2, sidx2, zeros, out, gi2, si2, rows, acc,
             semi, semg, sems):
        c = lax.axis_index("c")
        s = lax.axis_index("s")
        pltpu.sync_copy(zeros.at[pl.ds(s * stripe, stripe)],
                        acc.at[pl.ds(s * stripe, stripe)])
        plsc.subcore_barrier()
        chunk0 = s * n_chunks

        def run(table):
            @pl.loop(0, n_chunks // ibig)
            def _(ob):
                @pl.when(ob > 0)
                def _():
                    for b in range(ib):
                        pltpu.make_async_copy(
                            zeros.at[pl.ds(0, k)], rows.at[b], sems).wait()

                blk0 = chunk0 + ob * ibig
                cg = pltpu.async_copy(gidx2.at[pl.ds(blk0, ibig)], gi2, semi)
                cs = pltpu.async_copy(sidx2.at[pl.ds(blk0, ibig)], si2, semi)
                cg.wait()
                cs.wait()

                @pl.loop(0, ibig // ib)
                def _(g):
                    @pl.when(g > 0)
                    def _():
                        for b in range(ib):
                            pltpu.make_async_copy(
                                zeros.at[pl.ds(0, k)], rows.at[b],
                                sems).wait()

                    gs = [pltpu.async_copy(table.at[gi2.at[g * ib + b]],
                                           rows.at[b], semg)
                          for b in range(ib)]
                    for b in range(ib):
                        gs[b].wait()
                        pltpu.async_copy(rows.at[b],
                                         acc.at[si2.at[g * ib + b]],
                                         sems, add=True)

            for b in range(ib):
                pltpu.make_async_copy(
                    zeros.at[pl.ds(0, k)], rows.at[b], sems).wait()

        @pl.when(c == 0)
        def _():
            run(t0)

        @pl.when(c == 1)
        def _():
            run(t1)

        plsc.subcore_barrier()
        pltpu.sync_copy(acc.at[pl.ds(s * stripe, stripe)],
                        out.at[c, pl.ds(s * stripe, stripe)])

    return body


@functools.lru_cache(maxsize=None)
def _sc_linear_segsum_dual(ep, k, ib, ibig, acc_rows):
    n_chunks = ep // (16 * k)
    stripe = acc_rows // 16

    @functools.partial(
        pl.kernel, mesh=_mesh(), compiler_params=_SC_PARAMS,
        out_type=jax.ShapeDtypeStruct((2, acc_rows, 32), f32),
        scratch_types=[
            pltpu.VMEM((ibig, k), jnp.int32),
            pltpu.VMEM((ib * k, 32), f32),
            pltpu.VMEM_SHARED((acc_rows, 32), f32),
            pltpu.SemaphoreType.DMA,
        ],
    )
    def body(r0, r1, sidx2, zeros, out, si2, rows_v, acc, sem):
        c = lax.axis_index("c")
        s = lax.axis_index("s")
        pltpu.sync_copy(zeros.at[pl.ds(s * stripe, stripe)],
                        acc.at[pl.ds(s * stripe, stripe)])
        plsc.subcore_barrier()
        chunk0 = s * n_chunks

        def run(rows2d):
            @pl.loop(0, n_chunks // ibig)
            def _(ob):
                @pl.when(ob > 0)
                def _():
                    for b in range(ib):
                        pltpu.make_async_copy(
                            zeros.at[pl.ds(0, k)],
                            rows_v.at[pl.ds(b * k, k)], sem).wait()

                blk0 = chunk0 + ob * ibig
                pltpu.sync_copy(sidx2.at[pl.ds(blk0, ibig)], si2)

                @pl.loop(0, ibig // ib)
                def _(g):
                    @pl.when(g > 0)
                    def _():
                        for b in range(ib):
                            pltpu.make_async_copy(
                                zeros.at[pl.ds(0, k)],
                                rows_v.at[pl.ds(b * k, k)], sem).wait()

                    blk = blk0 + g * ib
                    pltpu.sync_copy(rows2d.at[pl.ds(blk * k, ib * k)], rows_v)
                    for b in range(ib):
                        pltpu.async_copy(rows_v.at[pl.ds(b * k, k)],
                                         acc.at[si2.at[g * ib + b]],
                                         sem, add=True)

            for b in range(ib):
                pltpu.make_async_copy(
                    zeros.at[pl.ds(0, k)],
                    rows_v.at[pl.ds(b * k, k)], sem).wait()

        @pl.when(c == 0)
        def _():
            run(r0)

        @pl.when(c == 1)
        def _():
            run(r1)

        plsc.subcore_barrier()
        pltpu.sync_copy(acc.at[pl.ds(s * stripe, stripe)],
                        out.at[c, pl.ds(s * stripe, stripe)])

    return body


def _t0_embed(x_p, WeT, be):
    def body(x_ref, w_ref, b_ref, h0_ref, h1_ref):
        h = jnp.dot(x_ref[...], w_ref[...], preferred_element_type=f32) + b_ref[...]
        h0_ref[...] = h[:, :32]
        h1_ref[...] = h[:, 32:]

    return pl.pallas_call(
        body,
        grid=(NP // NPB,),
        in_specs=[
            pl.BlockSpec((NPB, 92), lambda i: (i, 0)),
            pl.BlockSpec((92, 64), lambda i: (0, 0)),
            pl.BlockSpec((1, 64), lambda i: (0, 0)),
        ],
        out_specs=[pl.BlockSpec((NPB, 32), lambda i: (i, 0))] * 2,
        out_shape=[jax.ShapeDtypeStruct((NP, 32), f32)] * 2,
    )(x_p, WeT, be)


def _t1a(hs, cnt_hn, ha_p, W1, b1):
    nb = NP // NPB

    def body(hs_ref, cnt_ref, ha_ref, w_ref, b_ref,
             z_ref, st_ref, acc):
        i = pl.program_id(0)
        r = 1.0 / jnp.maximum(cnt_ref[0, :, :1], 1.0)
        hm0 = hs_ref[0] * r
        hm1 = hs_ref[1] * r
        msg = jnp.concatenate([hm0, hm1, ha_ref[...]], axis=1)
        z = jnp.dot(msg, w_ref[...], preferred_element_type=f32) + b_ref[...]
        z_ref[...] = z
        rows = i * NPB + lax.broadcasted_iota(jnp.int32, (NPB, 1), 0)
        zm = jnp.where(rows < N, z, 0.0)
        s1 = jnp.sum(zm, axis=0)
        s2 = jnp.sum(zm * zm, axis=0)
        upd = jnp.concatenate(
            [s1[None, :], s2[None, :], jnp.zeros((6, 70), f32)], axis=0)

        @pl.when(i == 0)
        def _():
            acc[...] = jnp.zeros_like(acc)

        acc[...] += upd

        @pl.when(i == nb - 1)
        def _():
            st_ref[...] = acc[...]

    return pl.pallas_call(
        body,
        grid=(nb,),
        in_specs=[
            pl.BlockSpec((2, NPB, 32), lambda i: (0, i, 0)),
            pl.BlockSpec((2, NPB, 32), lambda i: (0, i, 0)),
            pl.BlockSpec((NPB, 35), lambda i: (i, 0)),
            pl.BlockSpec((99, 70), lambda i: (0, 0)),
            pl.BlockSpec((1, 70), lambda i: (0, 0)),
        ],
        out_specs=[
            pl.BlockSpec((NPB, 70), lambda i: (i, 0)),
            pl.BlockSpec((8, 70), lambda i: (0, 0)),
        ],
        out_shape=[
            jax.ShapeDtypeStruct((NP, 70), f32),
            jax.ShapeDtypeStruct((8, 70), f32),
        ],
        scratch_shapes=[pltpu.VMEM((8, 70), f32)],
    )(hs, cnt_hn, ha_p, W1, b1)


def _t1b(z, st, g1, be1, h0, h1, WA, bA, WB):
    def body(z_ref, st_ref, g_ref, be_ref, h0_ref, h1_ref,
             wa_ref, ba_ref, wb_ref, afc_ref, bfc_ref):
        mean = st_ref[0, :] * (1.0 / N)
        var = st_ref[1, :] * (1.0 / N) - mean * mean
        scale = g_ref[0, :] * lax.rsqrt(var + EPS)
        zn = (z_ref[...] - mean[None, :]) * scale[None, :] + be_ref[...]
        ha = jax.nn.sigmoid(zn[:, :35]) * jax.nn.softplus(zn[:, 35:70])
        bfc_ref[...] = jnp.dot(ha, wb_ref[...], preferred_element_type=f32)
        h = jnp.concatenate([h0_ref[...], h1_ref[...]], axis=1)
        afc_ref[...] = jnp.dot(h, wa_ref[...],
                               preferred_element_type=f32) + ba_ref[...]

    return pl.pallas_call(
        body,
        grid=(NP // NPB,),
        in_specs=[
            pl.BlockSpec((NPB, 70), lambda i: (i, 0)),
            pl.BlockSpec((8, 70), lambda i: (0, 0)),
            pl.BlockSpec((1, 70), lambda i: (0, 0)),
            pl.BlockSpec((1, 70), lambda i: (0, 0)),
            pl.BlockSpec((NPB, 32), lambda i: (i, 0)),
            pl.BlockSpec((NPB, 32), lambda i: (i, 0)),
            pl.BlockSpec((64, 128), lambda i: (0, 0)),
            pl.BlockSpec((1, 128), lambda i: (0, 0)),
            pl.BlockSpec((35, 128), lambda i: (0, 0)),
        ],
        out_specs=[
            pl.BlockSpec((NPB, 128), lambda i: (i, 0)),
            pl.BlockSpec((NPB, 128), lambda i: (i, 0)),
        ],
        out_shape=[jax.ShapeDtypeStruct((NP, 128), f32)] * 2,
    )(z, st, g1, be1, h0, h1, WA, bA, WB)


def _t2_gate(An, Bh):
    rows = An.shape[0]

    def body(a_ref, b_ref, m0_ref, m1_ref):
        e = a_ref[...] + b_ref[...]
        m = jax.nn.sigmoid(e[:, :64]) * jax.nn.softplus(e[:, 64:])
        m0_ref[...] = m[:, :32]
        m1_ref[...] = m[:, 32:]

    return pl.pallas_call(
        body,
        grid=(rows // EPB,),
        in_specs=[
            pl.BlockSpec((EPB, 128), lambda i: (i, 0)),
            pl.BlockSpec((EPB, 128), lambda i: (i, 0)),
        ],
        out_specs=[pl.BlockSpec((EPB, 32), lambda i: (i, 0))] * 2,
        out_shape=[jax.ShapeDtypeStruct((rows, 32), f32)] * 2,
    )(An, Bh)


def _t_sum2(a, b):
    av = a.reshape(2, NP // 4, 128)
    bv = b.reshape(2, NP // 4, 128)
    rb = NP // 4 // 8

    def body(a_ref, b_ref, o_ref):
        o_ref[...] = a_ref[...] + b_ref[...]

    out = pl.pallas_call(
        body,
        grid=(8,),
        in_specs=[pl.BlockSpec((2, rb, 128), lambda i: (0, i, 0))] * 2,
        out_specs=pl.BlockSpec((2, rb, 128), lambda i: (0, i, 0)),
        out_shape=jax.ShapeDtypeStruct((2, NP // 4, 128), f32),
    )(av, bv)
    return out.reshape(2, NP, 32)


def _t3a(ns, cnt_hn):
    nb = NP // NPB

    def body(ns_ref, cnt_ref, nm_ref, st_ref, acc):
        i = pl.program_id(0)
        r = 1.0 / jnp.maximum(cnt_ref[1, :, :1], 1.0)
        nm = jnp.concatenate([ns_ref[0] * r, ns_ref[1] * r], axis=1)
        nm_ref[...] = nm
        rows = i * NPB + lax.broadcasted_iota(jnp.int32, (NPB, 1), 0)
        nmm = jnp.where(rows < N, nm, 0.0)
        s1 = jnp.sum(nmm, axis=0)
        s2 = jnp.sum(nmm * nmm, axis=0)
        upd = jnp.concatenate(
            [s1[None, :], s2[None, :], jnp.zeros((6, 64), f32)], axis=0)

        @pl.when(i == 0)
        def _():
            acc[...] = jnp.zeros_like(acc)

        acc[...] += upd

        @pl.when(i == nb - 1)
        def _():
            st_ref[...] = acc[...]

    return pl.pallas_call(
        body,
        grid=(nb,),
        in_specs=[
            pl.BlockSpec((2, NPB, 32), lambda i: (0, i, 0)),
            pl.BlockSpec((2, NPB, 32), lambda i: (0, i, 0)),
        ],
        out_specs=[
            pl.BlockSpec((NPB, 64), lambda i: (i, 0)),
            pl.BlockSpec((8, 64), lambda i: (0, 0)),
        ],
        out_shape=[
            jax.ShapeDtypeStruct((NP, 64), f32),
            jax.ShapeDtypeStruct((8, 64), f32),
        ],
        scratch_shapes=[pltpu.VMEM((8, 64), f32)],
    )(ns, cnt_hn)


def _t3b(nm, st, go, bo, h0, h1):
    def body(nm_ref, st_ref, g_ref, b_ref, h0_ref, h1_ref, o0_ref, o1_ref):
        mean = st_ref[0, :] * (1.0 / N)
        var = st_ref[1, :] * (1.0 / N) - mean * mean
        scale = g_ref[0, :] * lax.rsqrt(var + EPS)
        y = (nm_ref[...] - mean[None, :]) * scale[None, :] + b_ref[...]
        h = jnp.concatenate([h0_ref[...], h1_ref[...]], axis=1)
        hn = jax.nn.relu(jax.nn.softplus(y + h))
        o0_ref[...] = hn[:, :32]
        o1_ref[...] = hn[:, 32:]

    return pl.pallas_call(
        body,
        grid=(NP // NPB,),
        in_specs=[
            pl.BlockSpec((NPB, 64), lambda i: (i, 0)),
            pl.BlockSpec((8, 64), lambda i: (0, 0)),
            pl.BlockSpec((1, 64), lambda i: (0, 0)),
            pl.BlockSpec((1, 64), lambda i: (0, 0)),
            pl.BlockSpec((NPB, 32), lambda i: (i, 0)),
            pl.BlockSpec((NPB, 32), lambda i: (i, 0)),
        ],
        out_specs=[pl.BlockSpec((NPB, 32), lambda i: (i, 0))] * 2,
        out_shape=[jax.ShapeDtypeStruct((NP, 32), f32)] * 2,
    )(nm, st, go, bo, h0, h1)


def _t4_head(gs, cnt_g, W2, b2, Wo, bo):
    def body(gs_ref, cnt_ref, w2_ref, b2_ref, wo_ref, bo_ref, o_ref):
        c = cnt_ref[0, :, :1] + cnt_ref[1, :, :1]
        r = 1.0 / jnp.maximum(c, 1.0)
        g = jnp.concatenate([gs_ref[0] * r, gs_ref[1] * r], axis=1)
        t = jax.nn.softplus(
            jnp.dot(g, w2_ref[...], preferred_element_type=f32) + b2_ref[...])
        o_ref[...] = jnp.dot(t, wo_ref[...],
                             preferred_element_type=f32) + bo_ref[...]

    return pl.pallas_call(
        body,
        in_specs=[
            pl.BlockSpec((2, GACC, 32), lambda: (0, 0, 0)),
            pl.BlockSpec((2, GACC, 32), lambda: (0, 0, 0)),
            pl.BlockSpec((64, 128), lambda: (0, 0)),
            pl.BlockSpec((1, 128), lambda: (0, 0)),
            pl.BlockSpec((128, 128), lambda: (0, 0)),
            pl.BlockSpec((1, 128), lambda: (0, 0)),
        ],
        out_specs=pl.BlockSpec((GACC, 128), lambda: (0, 0)),
        out_shape=jax.ShapeDtypeStruct((GACC, 128), f32),
    )(gs, cnt_g, W2, b2, Wo, bo)


def kernel(x, hyperedge_index, hedge_attr, batch, params):
    nidx = hyperedge_index[0]
    hidx = hyperedge_index[1]
    nidx_g = jnp.pad(nidx, (0, EP - E)).reshape(EP // 128, 128)
    nidx_s = jnp.pad(nidx, (0, EP - E),
                     constant_values=N).reshape(EP // 128, 128)
    hidx_g = jnp.pad(hidx, (0, EP - E)).reshape(EP // 128, 128)
    hidx_s = jnp.pad(hidx, (0, EP - E),
                     constant_values=N).reshape(EP // 128, 128)
    batch_s = jnp.pad(batch, (0, NP - N),
                      constant_values=G).reshape(NP // 32, 32)
    nidx_g100 = jnp.pad(nidx, (0, EP - E)).reshape(EP // 100, 100)
    hidx_g100 = jnp.pad(hidx, (0, EP - E)).reshape(EP // 100, 100)

    x_p = jnp.pad(x, ((0, NP - N), (0, 0)))
    ha_p = jnp.pad(hedge_attr, ((0, NP - N), (0, 0)))
    zeros_np = jnp.zeros((NP, 32), f32)
    ones_e = jnp.ones((128, 32), f32)
    ones_g = jnp.ones((32, 32), f32)

    p = params
    WeT = p['embed']['W'].T
    be = p['embed']['b'][None, :]

    cnt_hn = _sc_count_dual(EP, 128, 4, 20, NP)(hidx_s, nidx_s,
                                                ones_e, zeros_np)
    cnt_g = _sc_count(NP, 32, 7, 49, GACC)(batch_s, ones_g, zeros_np)

    h0, h1 = _t0_embed(x_p, WeT, be)

    for lp in p['layers']:
        W1 = jnp.concatenate([lp['lin_f1']['W'].T, lp['lin_c1']['W'].T], axis=1)
        b1 = jnp.concatenate([lp['lin_f1']['b'], lp['lin_c1']['b']])[None, :]
        g1 = jnp.concatenate([lp['bn_f']['g'], lp['bn_c']['g']])[None, :]
        be1 = jnp.concatenate([lp['bn_f']['b'], lp['bn_c']['b']])[None, :]
        WA = jnp.concatenate([lp['lin_f2']['W'][:, :64].T,
                              lp['lin_c2']['W'][:, :64].T], axis=1)
        bA = jnp.concatenate([lp['lin_f2']['b'], lp['lin_c2']['b']])[None, :]
        WB = jnp.concatenate([lp['lin_f2']['W'][:, 64:].T,
                              lp['lin_c2']['W'][:, 64:].T], axis=1)

        hs = _sc_gather_segsum_dual(EP, 128, 4, 20, NP)(h0, h1, nidx_g,
                                                        hidx_s, zeros_np)
        z, st = _t1a(hs, cnt_hn, ha_p, W1, b1)
        Afc, Bfc = _t1b(z, st, g1, be1, h0, h1, WA, bA, WB)
        EH = EP // 2
        HR = EH // 128
        H100 = EH // 100
        An0, Bh0 = _sc_gather2(EH, 100, 4, 16)(Afc, Bfc,
                                               nidx_g100[:H100],
                                               hidx_g100[:H100])
        An1, Bh1 = _sc_gather2(EH, 100, 4, 16)(Afc, Bfc,
                                               nidx_g100[H100:],
                                               hidx_g100[H100:])
        m00, m10 = _t2_gate(An0, Bh0)
        m01, m11 = _t2_gate(An1, Bh1)
        nsa = _sc_linear_segsum_dual(EH, 128, 4, 20, NP)(m00, m10,
                                                         nidx_s[:HR],
                                                         zeros_np)
        nsb = _sc_linear_segsum_dual(EH, 128, 4, 20, NP)(m01, m11,
                                                         nidx_s[HR:],
                                                         zeros_np)
        nm, nst = _t3a(_t_sum2(nsa, nsb), cnt_hn)
        h0, h1 = _t3b(nm, nst, lp['bn_o']['g'][None, :],
                      lp['bn_o']['b'][None, :], h0, h1)

    gs = _sc_linear_segsum_dual(NP, 32, 7, 14, GACC)(h0, h1, batch_s,
                                                     zeros_np)

    W2 = p['l2']['W'].T
    b2 = p['l2']['b'][None, :]
    Wo = jnp.zeros((128, 128), f32).at[:, :1].set(p['out']['W'].T)
    bo = jnp.broadcast_to(p['out']['b'][None, :], (1, 128))

    out = _t4_head(gs, cnt_g, W2, b2, Wo, bo)
    return out[:G, :1]


# speedup vs baseline: 1.3406x; 1.0074x over previous
import functools

import jax
import jax.numpy as jnp
from jax import lax
from jax.experimental import pallas as pl
from jax.experimental.pallas import tpu as pltpu
from jax.experimental.pallas import tpu_sc as plsc

N = 50000
NP = 50176
E = 800000
EP = 819200
G = 256
GACC = 272
EPS = 1e-5
NPB = 6272
EPB = 3200
f32 = jnp.float32

_mesh = lambda: plsc.VectorSubcoreMesh(core_axis_name="c", subcore_axis_name="s")
_SC_PARAMS = pltpu.CompilerParams(use_tc_tiling_on_sc=False)


@functools.lru_cache(maxsize=None)
def _sc_count(ep, k, ib, ibig, acc_rows):
    n_chunks = ep // (32 * k)
    stripe = acc_rows // 16

    @functools.partial(
        pl.kernel, mesh=_mesh(), compiler_params=_SC_PARAMS,
        out_type=jax.ShapeDtypeStruct((2, acc_rows, 32), f32),
        scratch_types=[
            pltpu.VMEM((ibig, k), jnp.int32),
            pltpu.VMEM((k, 32), f32),
            pltpu.VMEM_SHARED((acc_rows, 32), f32),
            pltpu.SemaphoreType.DMA,
        ],
    )
    def body(sidx2, ones, zeros, out, si2, ones_v, acc, sem):
        c = lax.axis_index("c")
        s = lax.axis_index("s")
        pltpu.sync_copy(zeros.at[pl.ds(s * stripe, stripe)],
                        acc.at[pl.ds(s * stripe, stripe)])
        pltpu.sync_copy(ones.at[pl.ds(0, k)], ones_v)
        plsc.subcore_barrier()
        chunk0 = (c * 16 + s) * n_chunks

        @pl.loop(0, n_chunks // ibig)
        def _(ob):
            blk0 = chunk0 + ob * ibig
            pltpu.sync_copy(sidx2.at[pl.ds(blk0, ibig)], si2)

            @pl.loop(0, ibig // ib)
            def _(g):
                cps = [pltpu.async_copy(ones_v, acc.at[si2.at[g * ib + b]],
                                        sem, add=True)
                       for b in range(ib)]
                for cp in cps:
                    cp.wait()

        plsc.subcore_barrier()
        pltpu.sync_copy(acc.at[pl.ds(s * stripe, stripe)],
                        out.at[c, pl.ds(s * stripe, stripe)])

    return body


@functools.lru_cache(maxsize=None)
def _sc_gather_segsum(ep, k, ib, ibig, acc_rows):
    n_chunks = ep // (32 * k)
    stripe = acc_rows // 16

    @functools.partial(
        pl.kernel, mesh=_mesh(), compiler_params=_SC_PARAMS,
        out_type=jax.ShapeDtypeStruct((2, acc_rows, 32), f32),
        scratch_types=[
            pltpu.VMEM((ibig, k), jnp.int32),
            pltpu.VMEM((ibig, k), jnp.int32),
            pltpu.VMEM((ib, k, 32), f32),
            pltpu.VMEM_SHARED((acc_rows, 32), f32),
            pltpu.SemaphoreType.DMA,
            pltpu.SemaphoreType.DMA,
            pltpu.SemaphoreType.DMA,
        ],
    )
    def body(table, gidx2, sidx2, zeros, out, gi2, si2, rows, acc,
             semi, semg, sems):
        c = lax.axis_index("c")
        s = lax.axis_index("s")
        pltpu.sync_copy(zeros.at[pl.ds(s * stripe, stripe)],
                        acc.at[pl.ds(s * stripe, stripe)])
        plsc.subcore_barrier()
        chunk0 = (c * 16 + s) * n_chunks
        n_inner = ibig // ib

        @pl.loop(0, n_chunks // ibig)
        def _(ob):
            @pl.when(ob > 0)
            def _():
                for b in range(ib):
                    pltpu.make_async_copy(
                        zeros.at[pl.ds(0, k)], rows.at[b], sems).wait()

            blk0 = chunk0 + ob * ibig
            cg = pltpu.async_copy(gidx2.at[pl.ds(blk0, ibig)], gi2, semi)
            cs = pltpu.async_copy(sidx2.at[pl.ds(blk0, ibig)], si2, semi)
            cg.wait()
            cs.wait()

            @pl.loop(0, n_inner)
            def _(g):
                @pl.when(g > 0)
                def _():
                    for b in range(ib):
                        pltpu.make_async_copy(
                            zeros.at[pl.ds(0, k)], rows.at[b], sems).wait()

                gs = [pltpu.async_copy(table.at[gi2.at[g * ib + b]],
                                       rows.at[b], semg)
                      for b in range(ib)]
                for b in range(ib):
                    gs[b].wait()
                    pltpu.async_copy(rows.at[b], acc.at[si2.at[g * ib + b]],
                                     sems, add=True)

        for b in range(ib):
            pltpu.make_async_copy(
                zeros.at[pl.ds(0, k)], rows.at[b], sems).wait()
        plsc.subcore_barrier()
        pltpu.sync_copy(acc.at[pl.ds(s * stripe, stripe)],
                        out.at[c, pl.ds(s * stripe, stripe)])

    return body


@functools.lru_cache(maxsize=None)
def _sc_linear_segsum(ep, k, ib, ibig, acc_rows):
    n_chunks = ep // (32 * k)
    stripe = acc_rows // 16

    @functools.partial(
        pl.kernel, mesh=_mesh(), compiler_params=_SC_PARAMS,
        out_type=jax.ShapeDtypeStruct((2, acc_rows, 32), f32),
        scratch_types=[
            pltpu.VMEM((ibig, k), jnp.int32),
            pltpu.VMEM((ib * k, 32), f32),
            pltpu.VMEM_SHARED((acc_rows, 32), f32),
            pltpu.SemaphoreType.DMA,
        ],
    )
    def body(rows2d, sidx2, zeros, out, si2, rows_v, acc, sem):
        c = lax.axis_index("c")
        s = lax.axis_index("s")
        pltpu.sync_copy(zeros.at[pl.ds(s * stripe, stripe)],
                        acc.at[pl.ds(s * stripe, stripe)])
        plsc.subcore_barrier()
        chunk0 = (c * 16 + s) * n_chunks

        @pl.loop(0, n_chunks // ibig)
        def _(ob):
            @pl.when(ob > 0)
            def _():
                for b in range(ib):
                    pltpu.make_async_copy(
                        zeros.at[pl.ds(0, k)],
                        rows_v.at[pl.ds(b * k, k)], sem).wait()

            blk0 = chunk0 + ob * ibig
            pltpu.sync_copy(sidx2.at[pl.ds(blk0, ibig)], si2)

            @pl.loop(0, ibig // ib)
            def _(g):
                @pl.when(g > 0)
                def _():
                    for b in range(ib):
                        pltpu.make_async_copy(
                            zeros.at[pl.ds(0, k)],
                            rows_v.at[pl.ds(b * k, k)], sem).wait()

                blk = blk0 + g * ib
                pltpu.sync_copy(rows2d.at[pl.ds(blk * k, ib * k)], rows_v)
                for b in range(ib):
                    pltpu.async_copy(rows_v.at[pl.ds(b * k, k)],
                                     acc.at[si2.at[g * ib + b]],
                                     sem, add=True)

        for b in range(ib):
            pltpu.make_async_copy(
                zeros.at[pl.ds(0, k)], rows_v.at[pl.ds(b * k, k)], sem).wait()
        plsc.subcore_barrier()
        pltpu.sync_copy(acc.at[pl.ds(s * stripe, stripe)],
                        out.at[c, pl.ds(s * stripe, stripe)])

    return body


@functools.lru_cache(maxsize=None)
def _sc_gather2(ep, k, ib, ibig):
    n_chunks = ep // (32 * k)

    @functools.partial(
        pl.kernel, mesh=_mesh(), compiler_params=_SC_PARAMS,
        out_type=(jax.ShapeDtypeStruct((ep, 128), f32),
                  jax.ShapeDtypeStruct((ep, 128), f32)),
        scratch_types=[
            pltpu.VMEM((ibig, k), jnp.int32),
            pltpu.VMEM((ibig, k), jnp.int32),
            pltpu.VMEM((ib, k, 128), f32),
            pltpu.VMEM((ib, k, 128), f32),
            pltpu.SemaphoreType.DMA,
            pltpu.SemaphoreType.DMA,
            pltpu.SemaphoreType.DMA,
        ],
    )
    def body(tA, tB, idxA2, idxB2, outA, outB, ia2, ib2, bufA, bufB,
             semi, semg, semw):
        c = lax.axis_index("c")
        s = lax.axis_index("s")
        chunk0 = (c * 16 + s) * n_chunks

        def drain_writes():
            for b in range(ib):
                pltpu.make_async_copy(
                    tA.at[pl.ds(0, k)], bufA.at[b], semw).wait()
                pltpu.make_async_copy(
                    tA.at[pl.ds(0, k)], bufB.at[b], semw).wait()

        @pl.loop(0, n_chunks // ibig)
        def _(ob):
            @pl.when(ob > 0)
            def _():
                drain_writes()

            blk0 = chunk0 + ob * ibig
            ca = pltpu.async_copy(idxA2.at[pl.ds(blk0, ibig)], ia2, semi)
            cb = pltpu.async_copy(idxB2.at[pl.ds(blk0, ibig)], ib2, semi)
            ca.wait()
            cb.wait()

            @pl.loop(0, ibig // ib)
            def _(g):
                @pl.when(g > 0)
                def _():
                    drain_writes()

                blk = blk0 + g * ib
                gs = []
                for b in range(ib):
                    gs.append(pltpu.async_copy(
                        tA.at[ia2.at[g * ib + b]], bufA.at[b], semg))
                    gs.append(pltpu.async_copy(
                        tB.at[ib2.at[g * ib + b]], bufB.at[b], semg))
                for b in range(ib):
                    gs[2 * b].wait()
                    pltpu.async_copy(
                        bufA.at[b], outA.at[pl.ds((blk + b) * k, k)], semw)
                    gs[2 * b + 1].wait()
                    pltpu.async_copy(
                        bufB.at[b], outB.at[pl.ds((blk + b) * k, k)], semw)

        drain_writes()

    return body


@functools.lru_cache(maxsize=None)
def _sc_count_dual(ep, k, ib, ibig, acc_rows):
    n_chunks = ep // (16 * k)
    stripe = acc_rows // 16

    @functools.partial(
        pl.kernel, mesh=_mesh(), compiler_params=_SC_PARAMS,
        out_type=jax.ShapeDtypeStruct((2, acc_rows, 32), f32),
        scratch_types=[
            pltpu.VMEM((ibig, k), jnp.int32),
            pltpu.VMEM((k, 32), f32),
            pltpu.VMEM_SHARED((acc_rows, 32), f32),
            pltpu.SemaphoreType.DMA,
        ],
    )
    def body(sidx2a, sidx2b, ones, zeros, out, si2, ones_v, acc, sem):
        c = lax.axis_index("c")
        s = lax.axis_index("s")
        pltpu.sync_copy(zeros.at[pl.ds(s * stripe, stripe)],
                        acc.at[pl.ds(s * stripe, stripe)])
        pltpu.sync_copy(ones.at[pl.ds(0, k)], ones_v)
        plsc.subcore_barrier()
        chunk0 = s * n_chunks

        def run(sidx2):
            @pl.loop(0, n_chunks // ibig)
            def _(ob):
                blk0 = chunk0 + ob * ibig
                pltpu.sync_copy(sidx2.at[pl.ds(blk0, ibig)], si2)

                @pl.loop(0, ibig // ib)
                def _(g):
                    cps = [pltpu.async_copy(ones_v,
                                            acc.at[si2.at[g * ib + b]],
                                            sem, add=True)
                           for b in range(ib)]
                    for cp in cps:
                        cp.wait()

        @pl.when(c == 0)
        def _():
            run(sidx2a)

        @pl.when(c == 1)
        def _():
            run(sidx2b)

        plsc.subcore_barrier()
        pltpu.sync_copy(acc.at[pl.ds(s * stripe, stripe)],
                        out.at[c, pl.ds(s * stripe, stripe)])

    return body


@functools.lru_cache(maxsize=None)
def _sc_gather_segsum_dual(ep, k, ib, ibig, acc_rows):
    n_chunks = ep // (16 * k)
    stripe = acc_rows // 16

    @functools.partial(
        pl.kernel, mesh=_mesh(), compiler_params=_SC_PARAMS,
        out_type=jax.ShapeDtypeStruct((2, acc_rows, 32), f32),
        scratch_types=[
            pltpu.VMEM((ibig, k), jnp.int32),
            pltpu.VMEM((ibig, k), jnp.int32),
            pltpu.VMEM((ib, k, 32), f32),
            pltpu.VMEM_SHARED((acc_rows, 32), f32),
            pltpu.SemaphoreType.DMA,
            pltpu.SemaphoreType.DMA,
            pltpu.SemaphoreType.DMA,
        ],
    )
    def body(t0, t1, gidx2, sidx2, zeros, out, gi2, si2, rows, acc,
             semi, semg, sems):
        c = lax.axis_index("c")
        s = lax.axis_index("s")
        pltpu.sync_copy(zeros.at[pl.ds(s * stripe, stripe)],
                        acc.at[pl.ds(s * stripe, stripe)])
        plsc.subcore_barrier()
        chunk0 = s * n_chunks

        def run(table):
            @pl.loop(0, n_chunks // ibig)
            def _(ob):
                @pl.when(ob > 0)
                def _():
                    for b in range(ib):
                        pltpu.make_async_copy(
                            zeros.at[pl.ds(0, k)], rows.at[b], sems).wait()

                blk0 = chunk0 + ob * ibig
                cg = pltpu.async_copy(gidx2.at[pl.ds(blk0, ibig)], gi2, semi)
                cs = pltpu.async_copy(sidx2.at[pl.ds(blk0, ibig)], si2, semi)
                cg.wait()
                cs.wait()

                @pl.loop(0, ibig // ib)
                def _(g):
                    @pl.when(g > 0)
                    def _():
                        for b in range(ib):
                            pltpu.make_async_copy(
                                zeros.at[pl.ds(0, k)], rows.at[b],
                                sems).wait()

                    gs = [pltpu.async_copy(table.at[gi2.at[g * ib + b]],
                                           rows.at[b], semg)
                          for b in range(ib)]
                    for b in range(ib):
                        gs[b].wait()
                        pltpu.async_copy(rows.at[b],
                                         acc.at[si2.at[g * ib + b]],
                                         sems, add=True)

            for b in range(ib):
                pltpu.make_async_copy(
                    zeros.at[pl.ds(0, k)], rows.at[b], sems).wait()

        @pl.when(c == 0)
        def _():
            run(t0)

        @pl.when(c == 1)
        def _():
            run(t1)

        plsc.subcore_barrier()
        pltpu.sync_copy(acc.at[pl.ds(s * stripe, stripe)],
                        out.at[c, pl.ds(s * stripe, stripe)])

    return body


@functools.lru_cache(maxsize=None)
def _sc_linear_segsum_dual(ep, k, ib, ibig, acc_rows):
    n_chunks = ep // (16 * k)
    stripe = acc_rows // 16

    @functools.partial(
        pl.kernel, mesh=_mesh(), compiler_params=_SC_PARAMS,
        out_type=jax.ShapeDtypeStruct((2, acc_rows, 32), f32),
        scratch_types=[
            pltpu.VMEM((ibig, k), jnp.int32),
            pltpu.VMEM((ib * k, 32), f32),
            pltpu.VMEM_SHARED((acc_rows, 32), f32),
            pltpu.SemaphoreType.DMA,
        ],
    )
    def body(r0, r1, sidx2, zeros, out, si2, rows_v, acc, sem):
        c = lax.axis_index("c")
        s = lax.axis_index("s")
        pltpu.sync_copy(zeros.at[pl.ds(s * stripe, stripe)],
                        acc.at[pl.ds(s * stripe, stripe)])
        plsc.subcore_barrier()
        chunk0 = s * n_chunks

        def run(rows2d):
            @pl.loop(0, n_chunks // ibig)
            def _(ob):
                @pl.when(ob > 0)
                def _():
                    for b in range(ib):
                        pltpu.make_async_copy(
                            zeros.at[pl.ds(0, k)],
                            rows_v.at[pl.ds(b * k, k)], sem).wait()

                blk0 = chunk0 + ob * ibig
                pltpu.sync_copy(sidx2.at[pl.ds(blk0, ibig)], si2)

                @pl.loop(0, ibig // ib)
                def _(g):
                    @pl.when(g > 0)
                    def _():
                        for b in range(ib):
                            pltpu.make_async_copy(
                                zeros.at[pl.ds(0, k)],
                                rows_v.at[pl.ds(b * k, k)], sem).wait()

                    blk = blk0 + g * ib
                    pltpu.sync_copy(rows2d.at[pl.ds(blk * k, ib * k)], rows_v)
                    for b in range(ib):
                        pltpu.async_copy(rows_v.at[pl.ds(b * k, k)],
                                         acc.at[si2.at[g * ib + b]],
                                         sem, add=True)

            for b in range(ib):
                pltpu.make_async_copy(
                    zeros.at[pl.ds(0, k)],
                    rows_v.at[pl.ds(b * k, k)], sem).wait()

        @pl.when(c == 0)
        def _():
            run(r0)

        @pl.when(c == 1)
        def _():
            run(r1)

        plsc.subcore_barrier()
        pltpu.sync_copy(acc.at[pl.ds(s * stripe, stripe)],
                        out.at[c, pl.ds(s * stripe, stripe)])

    return body


def _t0_embed(x_p, WeT, be):
    def body(x_ref, w_ref, b_ref, h0_ref, h1_ref):
        h = jnp.dot(x_ref[...], w_ref[...], preferred_element_type=f32) + b_ref[...]
        h0_ref[...] = h[:, :32]
        h1_ref[...] = h[:, 32:]

    return pl.pallas_call(
        body,
        grid=(NP // NPB,),
        in_specs=[
            pl.BlockSpec((NPB, 92), lambda i: (i, 0)),
            pl.BlockSpec((92, 64), lambda i: (0, 0)),
            pl.BlockSpec((1, 64), lambda i: (0, 0)),
        ],
        out_specs=[pl.BlockSpec((NPB, 32), lambda i: (i, 0))] * 2,
        out_shape=[jax.ShapeDtypeStruct((NP, 32), f32)] * 2,
    )(x_p, WeT, be)


def _t1a(hs, cnt_hn, ha_p, W1, b1):
    nb = NP // NPB

    def body(hs_ref, cnt_ref, ha_ref, w_ref, b_ref,
             z_ref, st_ref, acc):
        i = pl.program_id(0)
        r = 1.0 / jnp.maximum(cnt_ref[0, :, :1], 1.0)
        hm0 = hs_ref[0] * r
        hm1 = hs_ref[1] * r
        msg = jnp.concatenate([hm0, hm1, ha_ref[...]], axis=1)
        z = jnp.dot(msg, w_ref[...], preferred_element_type=f32) + b_ref[...]
        z_ref[...] = z
        rows = i * NPB + lax.broadcasted_iota(jnp.int32, (NPB, 1), 0)
        zm = jnp.where(rows < N, z, 0.0)
        s1 = jnp.sum(zm, axis=0)
        s2 = jnp.sum(zm * zm, axis=0)
        upd = jnp.concatenate(
            [s1[None, :], s2[None, :], jnp.zeros((6, 70), f32)], axis=0)

        @pl.when(i == 0)
        def _():
            acc[...] = jnp.zeros_like(acc)

        acc[...] += upd

        @pl.when(i == nb - 1)
        def _():
            st_ref[...] = acc[...]

    return pl.pallas_call(
        body,
        grid=(nb,),
        in_specs=[
            pl.BlockSpec((2, NPB, 32), lambda i: (0, i, 0)),
            pl.BlockSpec((2, NPB, 32), lambda i: (0, i, 0)),
            pl.BlockSpec((NPB, 35), lambda i: (i, 0)),
            pl.BlockSpec((99, 70), lambda i: (0, 0)),
            pl.BlockSpec((1, 70), lambda i: (0, 0)),
        ],
        out_specs=[
            pl.BlockSpec((NPB, 70), lambda i: (i, 0)),
            pl.BlockSpec((8, 70), lambda i: (0, 0)),
        ],
        out_shape=[
            jax.ShapeDtypeStruct((NP, 70), f32),
            jax.ShapeDtypeStruct((8, 70), f32),
        ],
        scratch_shapes=[pltpu.VMEM((8, 70), f32)],
    )(hs, cnt_hn, ha_p, W1, b1)


def _t1b(z, st, g1, be1, h0, h1, WA, bA, WB):
    def body(z_ref, st_ref, g_ref, be_ref, h0_ref, h1_ref,
             wa_ref, ba_ref, wb_ref, afc_ref, bfc_ref):
        mean = st_ref[0, :] * (1.0 / N)
        var = st_ref[1, :] * (1.0 / N) - mean * mean
        scale = g_ref[0, :] * lax.rsqrt(var + EPS)
        zn = (z_ref[...] - mean[None, :]) * scale[None, :] + be_ref[...]
        ha = jax.nn.sigmoid(zn[:, :35]) * jax.nn.softplus(zn[:, 35:70])
        bfc_ref[...] = jnp.dot(ha, wb_ref[...], preferred_element_type=f32)
        h = jnp.concatenate([h0_ref[...], h1_ref[...]], axis=1)
        afc_ref[...] = jnp.dot(h, wa_ref[...],
                               preferred_element_type=f32) + ba_ref[...]

    return pl.pallas_call(
        body,
        grid=(NP // NPB,),
        in_specs=[
            pl.BlockSpec((NPB, 70), lambda i: (i, 0)),
            pl.BlockSpec((8, 70), lambda i: (0, 0)),
            pl.BlockSpec((1, 70), lambda i: (0, 0)),
            pl.BlockSpec((1, 70), lambda i: (0, 0)),
            pl.BlockSpec((NPB, 32), lambda i: (i, 0)),
            pl.BlockSpec((NPB, 32), lambda i: (i, 0)),
            pl.BlockSpec((64, 128), lambda i: (0, 0)),
            pl.BlockSpec((1, 128), lambda i: (0, 0)),
            pl.BlockSpec((35, 128), lambda i: (0, 0)),
        ],
        out_specs=[
            pl.BlockSpec((NPB, 128), lambda i: (i, 0)),
            pl.BlockSpec((NPB, 128), lambda i: (i, 0)),
        ],
        out_shape=[jax.ShapeDtypeStruct((NP, 128), f32)] * 2,
    )(z, st, g1, be1, h0, h1, WA, bA, WB)


def _t2_gate(An, Bh):
    rows = An.shape[0]

    def body(a_ref, b_ref, m0_ref, m1_ref):
        e = a_ref[...] + b_ref[...]
        m = jax.nn.sigmoid(e[:, :64]) * jax.nn.softplus(e[:, 64:])
        m0_ref[...] = m[:, :32]
        m1_ref[...] = m[:, 32:]

    return pl.pallas_call(
        body,
        grid=(rows // EPB,),
        in_specs=[
            pl.BlockSpec((EPB, 128), lambda i: (i, 0)),
            pl.BlockSpec((EPB, 128), lambda i: (i, 0)),
        ],
        out_specs=[pl.BlockSpec((EPB, 32), lambda i: (i, 0))] * 2,
        out_shape=[jax.ShapeDtypeStruct((rows, 32), f32)] * 2,
    )(An, Bh)


def _t_sum2(a, b):
    av = a.reshape(2, NP // 4, 128)
    bv = b.reshape(2, NP // 4, 128)
    rb = NP // 4 // 8

    def body(a_ref, b_ref, o_ref):
        o_ref[...] = a_ref[...] + b_ref[...]

    out = pl.pallas_call(
        body,
        grid=(8,),
        in_specs=[pl.BlockSpec((2, rb, 128), lambda i: (0, i, 0))] * 2,
        out_specs=pl.BlockSpec((2, rb, 128), lambda i: (0, i, 0)),
        out_shape=jax.ShapeDtypeStruct((2, NP // 4, 128), f32),
    )(av, bv)
    return out.reshape(2, NP, 32)


def _t3a(ns, cnt_hn):
    nb = NP // NPB

    def body(ns_ref, cnt_ref, nm_ref, st_ref, acc):
        i = pl.program_id(0)
        r = 1.0 / jnp.maximum(cnt_ref[1, :, :1], 1.0)
        nm = jnp.concatenate([ns_ref[0] * r, ns_ref[1] * r], axis=1)
        nm_ref[...] = nm
        rows = i * NPB + lax.broadcasted_iota(jnp.int32, (NPB, 1), 0)
        nmm = jnp.where(rows < N, nm, 0.0)
        s1 = jnp.sum(nmm, axis=0)
        s2 = jnp.sum(nmm * nmm, axis=0)
        upd = jnp.concatenate(
            [s1[None, :], s2[None, :], jnp.zeros((6, 64), f32)], axis=0)

        @pl.when(i == 0)
        def _():
            acc[...] = jnp.zeros_like(acc)

        acc[...] += upd

        @pl.when(i == nb - 1)
        def _():
            st_ref[...] = acc[...]

    return pl.pallas_call(
        body,
        grid=(nb,),
        in_specs=[
            pl.BlockSpec((2, NPB, 32), lambda i: (0, i, 0)),
            pl.BlockSpec((2, NPB, 32), lambda i: (0, i, 0)),
        ],
        out_specs=[
            pl.BlockSpec((NPB, 64), lambda i: (i, 0)),
            pl.BlockSpec((8, 64), lambda i: (0, 0)),
        ],
        out_shape=[
            jax.ShapeDtypeStruct((NP, 64), f32),
            jax.ShapeDtypeStruct((8, 64), f32),
        ],
        scratch_shapes=[pltpu.VMEM((8, 64), f32)],
    )(ns, cnt_hn)


def _t3b(nm, st, go, bo, h0, h1):
    def body(nm_ref, st_ref, g_ref, b_ref, h0_ref, h1_ref, o0_ref, o1_ref):
        mean = st_ref[0, :] * (1.0 / N)
        var = st_ref[1, :] * (1.0 / N) - mean * mean
        scale = g_ref[0, :] * lax.rsqrt(var + EPS)
        y = (nm_ref[...] - mean[None, :]) * scale[None, :] + b_ref[...]
        h = jnp.concatenate([h0_ref[...], h1_ref[...]], axis=1)
        hn = jax.nn.relu(jax.nn.softplus(y + h))
        o0_ref[...] = hn[:, :32]
        o1_ref[...] = hn[:, 32:]

    return pl.pallas_call(
        body,
        grid=(NP // NPB,),
        in_specs=[
            pl.BlockSpec((NPB, 64), lambda i: (i, 0)),
            pl.BlockSpec((8, 64), lambda i: (0, 0)),
            pl.BlockSpec((1, 64), lambda i: (0, 0)),
            pl.BlockSpec((1, 64), lambda i: (0, 0)),
            pl.BlockSpec((NPB, 32), lambda i: (i, 0)),
            pl.BlockSpec((NPB, 32), lambda i: (i, 0)),
        ],
        out_specs=[pl.BlockSpec((NPB, 32), lambda i: (i, 0))] * 2,
        out_shape=[jax.ShapeDtypeStruct((NP, 32), f32)] * 2,
    )(nm, st, go, bo, h0, h1)


def _t4_head(gs, cnt_g, W2, b2, Wo, bo):
    def body(gs_ref, cnt_ref, w2_ref, b2_ref, wo_ref, bo_ref, o_ref):
        c = cnt_ref[0, :, :1] + cnt_ref[1, :, :1]
        r = 1.0 / jnp.maximum(c, 1.0)
        g = jnp.concatenate([gs_ref[0] * r, gs_ref[1] * r], axis=1)
        t = jax.nn.softplus(
            jnp.dot(g, w2_ref[...], preferred_element_type=f32) + b2_ref[...])
        o_ref[...] = jnp.dot(t, wo_ref[...],
                             preferred_element_type=f32) + bo_ref[...]

    return pl.pallas_call(
        body,
        in_specs=[
            pl.BlockSpec((2, GACC, 32), lambda: (0, 0, 0)),
            pl.BlockSpec((2, GACC, 32), lambda: (0, 0, 0)),
            pl.BlockSpec((64, 128), lambda: (0, 0)),
            pl.BlockSpec((1, 128), lambda: (0, 0)),
            pl.BlockSpec((128, 128), lambda: (0, 0)),
            pl.BlockSpec((1, 128), lambda: (0, 0)),
        ],
        out_specs=pl.BlockSpec((GACC, 128), lambda: (0, 0)),
        out_shape=jax.ShapeDtypeStruct((GACC, 128), f32),
    )(gs, cnt_g, W2, b2, Wo, bo)


def kernel(x, hyperedge_index, hedge_attr, batch, params):
    nidx = hyperedge_index[0]
    hidx = hyperedge_index[1]
    nidx_g = jnp.pad(nidx, (0, EP - E)).reshape(EP // 128, 128)
    nidx_s = jnp.pad(nidx, (0, EP - E),
                     constant_values=N).reshape(EP // 128, 128)
    hidx_g = jnp.pad(hidx, (0, EP - E)).reshape(EP // 128, 128)
    hidx_s = jnp.pad(hidx, (0, EP - E),
                     constant_values=N).reshape(EP // 128, 128)
    batch_s = jnp.pad(batch, (0, NP - N),
                      constant_values=G).reshape(NP // 32, 32)
    nidx_g100 = jnp.pad(nidx, (0, EP - E)).reshape(EP // 100, 100)
    hidx_g100 = jnp.pad(hidx, (0, EP - E)).reshape(EP // 100, 100)

    x_p = jnp.pad(x, ((0, NP - N), (0, 0)))
    ha_p = jnp.pad(hedge_attr, ((0, NP - N), (0, 0)))
    zeros_np = jnp.zeros((NP, 32), f32)
    ones_e = jnp.ones((128, 32), f32)
    ones_g = jnp.ones((32, 32), f32)

    p = params
    WeT = p['embed']['W'].T
    be = p['embed']['b'][None, :]

    cnt_hn = _sc_count_dual(EP, 128, 4, 20, NP)(hidx_s, nidx_s,
                                                ones_e, zeros_np)
    cnt_g = _sc_count(NP, 32, 7, 49, GACC)(batch_s, ones_g, zeros_np)

    h0, h1 = _t0_embed(x_p, WeT, be)

    for lp in p['layers']:
        W1 = jnp.concatenate([lp['lin_f1']['W'].T, lp['lin_c1']['W'].T], axis=1)
        b1 = jnp.concatenate([lp['lin_f1']['b'], lp['lin_c1']['b']])[None, :]
        g1 = jnp.concatenate([lp['bn_f']['g'], lp['bn_c']['g']])[None, :]
        be1 = jnp.concatenate([lp['bn_f']['b'], lp['bn_c']['b']])[None, :]
        WA = jnp.concatenate([lp['lin_f2']['W'][:, :64].T,
                              lp['lin_c2']['W'][:, :64].T], axis=1)
        bA = jnp.concatenate([lp['lin_f2']['b'], lp['lin_c2']['b']])[None, :]
        WB = jnp.concatenate([lp['lin_f2']['W'][:, 64:].T,
                              lp['lin_c2']['W'][:, 64:].T], axis=1)

        hs = _sc_gather_segsum_dual(EP, 128, 5, 20, NP)(h0, h1, nidx_g,
                                                        hidx_s, zeros_np)
        z, st = _t1a(hs, cnt_hn, ha_p, W1, b1)
        Afc, Bfc = _t1b(z, st, g1, be1, h0, h1, WA, bA, WB)
        EH = EP // 2
        HR = EH // 128
        H100 = EH // 100
        An0, Bh0 = _sc_gather2(EH, 100, 4, 16)(Afc, Bfc,
                                               nidx_g100[:H100],
                                               hidx_g100[:H100])
        An1, Bh1 = _sc_gather2(EH, 100, 4, 16)(Afc, Bfc,
                                               nidx_g100[H100:],
                                               hidx_g100[H100:])
        m00, m10 = _t2_gate(An0, Bh0)
        m01, m11 = _t2_gate(An1, Bh1)
        nsa = _sc_linear_segsum_dual(EH, 128, 5, 20, NP)(m00, m10,
                                                         nidx_s[:HR],
                                                         zeros_np)
        nsb = _sc_linear_segsum_dual(EH, 128, 5, 20, NP)(m01, m11,
                                                         nidx_s[HR:],
                                                         zeros_np)
        nm, nst = _t3a(_t_sum2(nsa, nsb), cnt_hn)
        h0, h1 = _t3b(nm, nst, lp['bn_o']['g'][None, :],
                      lp['bn_o']['b'][None, :], h0, h1)

    gs = _sc_linear_segsum_dual(NP, 32, 7, 14, GACC)(h0, h1, batch_s,
                                                     zeros_np)

    W2 = p['l2']['W'].T
    b2 = p['l2']['b'][None, :]
    Wo = jnp.zeros((128, 128), f32).at[:, :1].set(p['out']['W'].T)
    bo = jnp.broadcast_to(p['out']['b'][None, :], (1, 128))

    out = _t4_head(gs, cnt_g, W2, b2, Wo, bo)
    return out[:G, :1]
